# baseline (speedup 1.0000x reference)
import functools

import jax
import jax.numpy as jnp
from jax import lax
from jax.experimental import pallas as pl
from jax.experimental.pallas import tpu as pltpu
from jax.experimental.pallas import tpu_sc as plsc

N_NODES = 10000
N_EDGES = 160000

NC = 2
NS = 16
NW = NC * NS
NPW = 320
NPAD = NW * NPW

K_FLUSH = 4096
G = 128
ECAP = N_EDGES + K_FLUSH + 256
SCH = 8000
BUFCAP = K_FLUSH + 192

_NEG_INF = float("-inf")


def _worker_id():
    return lax.axis_index("s") * NC + lax.axis_index("c")


def _sc_mesh():
    return plsc.VectorSubcoreMesh(
        core_axis_name="c", subcore_axis_name="s",
        num_cores=NC, num_subcores=NS)


@functools.lru_cache(maxsize=None)
def _get_bin_kernel():
    @functools.partial(
        pl.kernel,
        out_type=[
            jax.ShapeDtypeStruct((NW * ECAP,), jnp.int32),
            jax.ShapeDtypeStruct((NW * ECAP,), jnp.int32),
            jax.ShapeDtypeStruct((NW * 16,), jnp.int32),
        ],
        mesh=_sc_mesh(),
        scratch_types=[
            pltpu.VMEM((SCH,), jnp.int32),
            pltpu.VMEM((SCH,), jnp.int32),
            pltpu.VMEM((SCH,), jnp.int32),
            pltpu.VMEM((SCH,), jnp.int32),
            pltpu.VMEM((BUFCAP,), jnp.int32),
            pltpu.VMEM((BUFCAP,), jnp.int32),
            pltpu.VMEM((16,), jnp.int32),
            pltpu.SemaphoreType.DMA,
            pltpu.SemaphoreType.DMA,
        ],
        compiler_params=pltpu.CompilerParams(needs_layout_passes=False),
    )
    def bin_edges(src_hbm, dst_hbm, bsrc_hbm, bloc_hbm, cnt_hbm,
                  stage_sa, stage_da, stage_sb, stage_db,
                  buf_s, buf_l, cnt_v, sem_a, sem_b):
        w = _worker_id()
        lo = w * NPW

        def do_flush(pos, flushed):
            off = pl.multiple_of(w * ECAP + flushed, 8)
            pltpu.sync_copy(buf_s.at[pl.ds(0, K_FLUSH)],
                            bsrc_hbm.at[pl.ds(off, K_FLUSH)])
            pltpu.sync_copy(buf_l.at[pl.ds(0, K_FLUSH)],
                            bloc_hbm.at[pl.ds(off, K_FLUSH)])
            ts = buf_s[pl.ds(K_FLUSH, 16)]
            tl = buf_l[pl.ds(K_FLUSH, 16)]
            buf_s[pl.ds(0, 16)] = ts
            buf_l[pl.ds(0, 16)] = tl
            return pos - K_FLUSH, flushed + K_FLUSH

        def no_flush(pos, flushed):
            return pos, flushed

        lo_v = jnp.full((16,), lo, jnp.int32)
        hi_v = jnp.full((16,), lo + NPW, jnp.int32)
        zero_v = jnp.zeros((16,), jnp.int32)
        one_v = jnp.full((16,), 1, jnp.int32)

        def make_append(ss, dd):
            def append_chunk(i, carry):
                pos, flushed = carry
                d = dd[pl.ds(i * 16, 16)]
                s = ss[pl.ds(i * 16, 16)]
                m = (d >= lo_v) & (d < hi_v)
                csum = plsc.cumsum(jnp.where(m, one_v, zero_v))
                pos_v = jnp.full((16,), pos, jnp.int32)
                idxv = jnp.maximum(pos_v + csum - one_v, zero_v)
                plsc.store_scatter(buf_s, [idxv], s, mask=m)
                plsc.store_scatter(buf_l, [idxv], d - lo_v, mask=m)
                pos = pos + csum[15]
                return lax.cond(pos >= K_FLUSH, do_flush, no_flush,
                                pos, flushed)
            return append_chunk

        nb = N_EDGES // SCH
        bufs = [(stage_sa, stage_da, sem_a), (stage_sb, stage_db, sem_b)]

        def issue(cb, b):
            ss, dd, sem = bufs[b]
            pltpu.async_copy(src_hbm.at[pl.ds(cb * SCH, SCH)], ss, sem)
            pltpu.async_copy(dst_hbm.at[pl.ds(cb * SCH, SCH)], dd, sem)

        def drain(b):
            ss, dd, sem = bufs[b]
            pltpu.make_async_copy(src_hbm.at[pl.ds(0, SCH)], ss, sem).wait()
            pltpu.make_async_copy(src_hbm.at[pl.ds(0, SCH)], dd, sem).wait()

        issue(0, 0)
        carry = (jnp.int32(0), jnp.int32(0))
        for cb in range(nb):
            b = cb % 2
            drain(b)
            if cb + 1 < nb:
                issue(cb + 1, 1 - b)
            ss, dd, _ = bufs[b]
            carry = lax.fori_loop(0, SCH // 16, make_append(ss, dd), carry)
        pos, flushed = carry
        n_total = flushed + pos

        zeros16 = jnp.zeros((16,), jnp.int32)
        pad16 = jnp.full((16,), NPW, jnp.int32)
        for j in range(G // 16):
            buf_s[pl.ds(pos + j * 16, 16)] = zeros16
            buf_l[pl.ds(pos + j * 16, 16)] = pad16
        pos = pos + G
        pos, flushed = lax.cond(pos >= K_FLUSH, do_flush, no_flush,
                                pos, flushed)

        off = pl.multiple_of(w * ECAP + flushed, 8)
        pltpu.sync_copy(buf_s.at[pl.ds(0, K_FLUSH)],
                        bsrc_hbm.at[pl.ds(off, K_FLUSH)])
        pltpu.sync_copy(buf_l.at[pl.ds(0, K_FLUSH)],
                        bloc_hbm.at[pl.ds(off, K_FLUSH)])

        cnt_v[pl.ds(0, 16)] = jnp.full((16,), n_total, jnp.int32)
        pltpu.sync_copy(cnt_v, cnt_hbm.at[pl.ds(pl.multiple_of(w * 16, 8), 16)])

    return bin_edges


IB = 4096


@functools.lru_cache(maxsize=None)
def _get_segmax(C):
    g = 64 if C > 128 else 128
    cpb = IB // g
    nj = C // 16

    @functools.partial(
        pl.kernel,
        out_type=jax.ShapeDtypeStruct((nj * NPAD * 16,), jnp.float32),
        mesh=_sc_mesh(),
        scratch_types=(
            [pltpu.VMEM(((NPW + 1) * 16,), jnp.float32) for _ in range(nj)]
            + [
                pltpu.VMEM((g, C), jnp.float32),
                pltpu.VMEM((g, C), jnp.float32),
                pltpu.VMEM((IB,), jnp.int32),
                pltpu.VMEM((IB,), jnp.int32),
                pltpu.VMEM((16,), jnp.int32),
                pltpu.SemaphoreType.DMA,
                pltpu.SemaphoreType.DMA,
            ]
        ),
    )
    def seg_kernel(q_hbm, bsrc_hbm, bloc_hbm, cnt_hbm, s_hbm, *scr):
        accs = scr[:nj]
        rows_a, rows_b, ibuf_s, ibuf_l, cnt_v, sem_a, sem_b = scr[nj:]
        w = _worker_id()
        lo = w * NPW
        pltpu.sync_copy(cnt_hbm.at[pl.ds(pl.multiple_of(w * 16, 8), 16)],
                        cnt_v)
        n = cnt_v[pl.ds(0, 16)][0]
        nchunks = (n + (g - 1)) // g
        nblocks = (nchunks + (cpb - 1)) // cpb

        neg = jnp.full((16,), _NEG_INF, jnp.float32)

        def init_body(i, _):
            for j in range(nj):
                accs[j][pl.ds(i * 16, 16)] = neg
            return 0
        lax.fori_loop(0, NPW + 1, init_body, 0)

        def compute(rows, c):
            def group_body(gi, _):
                locv = ibuf_l[pl.ds(c * g + gi * 16, 16)]
                for t in range(16):
                    r16 = locv[t] * 16
                    i = gi * 16 + t
                    for j in range(nj):
                        sl = pl.ds(r16, 16)
                        accs[j][sl] = jnp.maximum(
                            accs[j][sl], rows[i, pl.ds(j * 16, 16)])
                return 0
            lax.fori_loop(0, g // 16, group_body, 0)

        def gather(c, rows, sem):
            pltpu.async_copy(q_hbm.at[ibuf_s.at[pl.ds(c * g, g)]], rows, sem)

        def wait(rows, sem):
            pltpu.make_async_copy(q_hbm.at[ibuf_s.at[pl.ds(0, g)]],
                                  rows, sem).wait()

        def block_body(ib, _):
            boff = pl.multiple_of(w * ECAP + ib * IB, 8)
            pltpu.sync_copy(bsrc_hbm.at[pl.ds(boff, IB)], ibuf_s)
            pltpu.sync_copy(bloc_hbm.at[pl.ds(boff, IB)], ibuf_l)
            ch = jnp.minimum(nchunks - ib * cpb, cpb)
            gather(0, rows_a, sem_a)

            def pair_body(p, _):
                c0 = 2 * p
                wait(rows_a, sem_a)

                @pl.when(c0 + 1 < ch)
                def _():
                    gather(c0 + 1, rows_b, sem_b)
                compute(rows_a, c0)

                @pl.when(c0 + 1 < ch)
                def _():
                    wait(rows_b, sem_b)

                    @pl.when(c0 + 2 < ch)
                    def _():
                        gather(c0 + 2, rows_a, sem_a)
                    compute(rows_b, c0 + 1)
                return 0
            lax.fori_loop(0, (ch + 1) // 2, pair_body, 0)
            return 0
        lax.fori_loop(0, nblocks, block_body, 0)

        for j in range(nj):
            pltpu.sync_copy(
                accs[j].at[pl.ds(0, NPW * 16)],
                s_hbm.at[pl.ds(pl.multiple_of(j * NPAD * 16 + lo * 16, 8),
                               NPW * 16)])

    return seg_kernel


def _unblock(s_blk, C):
    s3 = s_blk.reshape(C // 16, NPAD, 16)
    return jnp.transpose(s3, (1, 0, 2)).reshape(NPAD, C)[:N_NODES]


_TR = 1000


def _tc_first(x, A, bias, C, QW):
    cin = x.shape[1]

    def body(x_ref, a_ref, b_ref, p_ref, q_ref):
        r = jnp.dot(x_ref[...], a_ref[...],
                    preferred_element_type=jnp.float32) + b_ref[...]
        p_ref[...] = r[:, :C]
        q = r[:, C:]
        if QW > C:
            q = jnp.concatenate(
                [q, jnp.zeros((q.shape[0], QW - C), jnp.float32)], axis=1)
        q_ref[...] = q

    return pl.pallas_call(
        body,
        grid=(N_NODES // _TR,),
        in_specs=[
            pl.BlockSpec((_TR, cin), lambda i: (i, 0)),
            pl.BlockSpec((cin, 2 * C), lambda i: (0, 0)),
            pl.BlockSpec((1, 2 * C), lambda i: (0, 0)),
        ],
        out_specs=[
            pl.BlockSpec((_TR, C), lambda i: (i, 0)),
            pl.BlockSpec((_TR, QW), lambda i: (i, 0)),
        ],
        out_shape=[jax.ShapeDtypeStruct((N_NODES, C), jnp.float32),
                   jax.ShapeDtypeStruct((N_NODES, QW), jnp.float32)],
    )(x, A, bias)


def _tc_mid(p_prev, s_prev, A, bias, C):
    cin = p_prev.shape[1]

    def body(p_ref, s_ref, a_ref, b_ref, po_ref, qo_ref):
        xv = jnp.maximum(p_ref[...] + s_ref[...], 0.0)
        r = jnp.dot(xv, a_ref[...],
                    preferred_element_type=jnp.float32) + b_ref[...]
        po_ref[...] = r[:, :C]
        qo_ref[...] = r[:, C:]

    return pl.pallas_call(
        body,
        grid=(N_NODES // _TR,),
        in_specs=[
            pl.BlockSpec((_TR, cin), lambda i: (i, 0)),
            pl.BlockSpec((_TR, cin), lambda i: (i, 0)),
            pl.BlockSpec((cin, 2 * C), lambda i: (0, 0)),
            pl.BlockSpec((1, 2 * C), lambda i: (0, 0)),
        ],
        out_specs=[
            pl.BlockSpec((_TR, C), lambda i: (i, 0)),
            pl.BlockSpec((_TR, C), lambda i: (i, 0)),
        ],
        out_shape=[jax.ShapeDtypeStruct((N_NODES, C), jnp.float32)] * 2,
    )(p_prev, s_prev, A, bias)


def _tc_final(p3, s3a, s3b, x0, W4, b4, W5, b5):
    def body(p_ref, sa_ref, sb_ref, x0_ref, w4_ref, b4_ref, w5_ref, b5_ref,
             o_ref):
        s = jnp.concatenate([sa_ref[...], sb_ref[...]], axis=1)
        xv = jnp.maximum(p_ref[...] + s, 0.0)
        h = jnp.maximum(
            jnp.dot(xv, w4_ref[...], preferred_element_type=jnp.float32)
            + b4_ref[...], 0.0)
        o_ref[...] = (jnp.dot(h, w5_ref[...],
                              preferred_element_type=jnp.float32)
                      + b5_ref[...] + x0_ref[...])

    return pl.pallas_call(
        body,
        grid=(N_NODES // _TR,),
        in_specs=[
            pl.BlockSpec((_TR, 512), lambda i: (i, 0)),
            pl.BlockSpec((_TR, 256), lambda i: (i, 0)),
            pl.BlockSpec((_TR, 256), lambda i: (i, 0)),
            pl.BlockSpec((_TR, 3), lambda i: (i, 0)),
            pl.BlockSpec((512, 256), lambda i: (0, 0)),
            pl.BlockSpec((1, 256), lambda i: (0, 0)),
            pl.BlockSpec((256, 3), lambda i: (0, 0)),
            pl.BlockSpec((1, 3), lambda i: (0, 0)),
        ],
        out_specs=pl.BlockSpec((_TR, 3), lambda i: (i, 0)),
        out_shape=jax.ShapeDtypeStruct((N_NODES, 3), jnp.float32),
    )(p3, s3a, s3b, x0, W4, b4, W5, b5)


def _split_weights(W, b, cin):
    wa, wb = W[:cin], W[cin:]
    A = jnp.concatenate([wa - wb, wb], axis=1)
    bias = jnp.concatenate([b, jnp.zeros_like(b)])[None, :]
    return A, bias


def kernel(x, edge_index, W1, b1, W2, b2, W3, b3, W4, b4, W5, b5):
    src = edge_index[0]
    dst = edge_index[1]

    bsrc, bloc, counts = _get_bin_kernel()(src, dst)

    A1, bias1 = _split_weights(W1, b1, 3)
    A2, bias2 = _split_weights(W2, b2, 64)
    A3, bias3 = _split_weights(W3, b3, 128)

    P1, Q1 = _tc_first(x, A1, bias1, 64, 128)
    S1 = _unblock(_get_segmax(128)(Q1, bsrc, bloc, counts), 128)[:, :64]

    P2, Q2 = _tc_mid(P1, S1, A2, bias2, 128)
    S2 = _unblock(_get_segmax(128)(Q2, bsrc, bloc, counts), 128)

    P3, Q3 = _tc_mid(P2, S2, A3, bias3, 512)
    S3a = _unblock(_get_segmax(256)(Q3[:, :256], bsrc, bloc, counts), 256)
    S3b = _unblock(_get_segmax(256)(Q3[:, 256:], bsrc, bloc, counts), 256)

    return _tc_final(P3, S3a, S3b, x, W4, b4[None, :], W5, b5[None, :])


# speedup vs baseline: 1.7694x; 1.7694x over previous
import functools

import jax
import jax.numpy as jnp
from jax import lax
from jax.experimental import pallas as pl
from jax.experimental.pallas import tpu as pltpu
from jax.experimental.pallas import tpu_sc as plsc

N_NODES = 10000
N_EDGES = 160000

NC = 2
NS = 16
NW = NC * NS
NPW = 320
NPAD = NW * NPW

K_FLUSH = 4096
G = 128
ECAP = N_EDGES + K_FLUSH + 256
SCH = 8000
BUFCAP = K_FLUSH + 192
CAPV = 16384
OFFW = 352

_NEG_INF = float("-inf")


def _worker_id():
    return lax.axis_index("s") * NC + lax.axis_index("c")


def _sc_mesh():
    return plsc.VectorSubcoreMesh(
        core_axis_name="c", subcore_axis_name="s",
        num_cores=NC, num_subcores=NS)


@functools.lru_cache(maxsize=None)
def _get_bin_kernel():
    @functools.partial(
        pl.kernel,
        out_type=[
            jax.ShapeDtypeStruct((NW * ECAP,), jnp.int32),
            jax.ShapeDtypeStruct((NW * ECAP,), jnp.int32),
            jax.ShapeDtypeStruct((NW * 16,), jnp.int32),
            jax.ShapeDtypeStruct((NW * ECAP,), jnp.int32),
            jax.ShapeDtypeStruct((NW * OFFW,), jnp.int32),
        ],
        mesh=_sc_mesh(),
        scratch_types=[
            pltpu.VMEM((SCH,), jnp.int32),
            pltpu.VMEM((SCH,), jnp.int32),
            pltpu.VMEM((SCH,), jnp.int32),
            pltpu.VMEM((SCH,), jnp.int32),
            pltpu.VMEM((BUFCAP,), jnp.int32),
            pltpu.VMEM((BUFCAP,), jnp.int32),
            pltpu.VMEM((16,), jnp.int32),
            pltpu.VMEM((OFFW,), jnp.int32),
            pltpu.VMEM((OFFW,), jnp.int32),
            pltpu.VMEM((OFFW,), jnp.int32),
            pltpu.VMEM((CAPV,), jnp.int32),
            pltpu.SemaphoreType.DMA,
            pltpu.SemaphoreType.DMA,
        ],
        compiler_params=pltpu.CompilerParams(needs_layout_passes=False),
    )
    def bin_edges(src_hbm, dst_hbm, bsrc_hbm, bloc_hbm, cnt_hbm,
                  bsrt_hbm, offs_hbm,
                  stage_sa, stage_da, stage_sb, stage_db,
                  buf_s, buf_l, cnt_v, cnts_v, offs_v, offs2_v, sort_v,
                  sem_a, sem_b):
        w = _worker_id()
        lo = w * NPW

        def do_flush(pos, flushed):
            off = pl.multiple_of(w * ECAP + flushed, 8)
            pltpu.sync_copy(buf_s.at[pl.ds(0, K_FLUSH)],
                            bsrc_hbm.at[pl.ds(off, K_FLUSH)])
            pltpu.sync_copy(buf_l.at[pl.ds(0, K_FLUSH)],
                            bloc_hbm.at[pl.ds(off, K_FLUSH)])
            ts = buf_s[pl.ds(K_FLUSH, 16)]
            tl = buf_l[pl.ds(K_FLUSH, 16)]
            buf_s[pl.ds(0, 16)] = ts
            buf_l[pl.ds(0, 16)] = tl
            return pos - K_FLUSH, flushed + K_FLUSH

        def no_flush(pos, flushed):
            return pos, flushed

        lo_v = jnp.full((16,), lo, jnp.int32)
        hi_v = jnp.full((16,), lo + NPW, jnp.int32)
        zero_v = jnp.zeros((16,), jnp.int32)
        one_v = jnp.full((16,), 1, jnp.int32)

        def make_append(ss, dd):
            def append_chunk(i, carry):
                pos, flushed = carry
                d = dd[pl.ds(i * 16, 16)]
                s = ss[pl.ds(i * 16, 16)]
                m = (d >= lo_v) & (d < hi_v)
                csum = plsc.cumsum(jnp.where(m, one_v, zero_v))
                pos_v = jnp.full((16,), pos, jnp.int32)
                idxv = jnp.maximum(pos_v + csum - one_v, zero_v)
                plsc.store_scatter(buf_s, [idxv], s, mask=m)
                plsc.store_scatter(buf_l, [idxv], d - lo_v, mask=m)
                pos = pos + csum[15]
                return lax.cond(pos >= K_FLUSH, do_flush, no_flush,
                                pos, flushed)
            return append_chunk

        nb = N_EDGES // SCH
        bufs = [(stage_sa, stage_da, sem_a), (stage_sb, stage_db, sem_b)]

        def issue(cb, b):
            ss, dd, sem = bufs[b]
            pltpu.async_copy(src_hbm.at[pl.ds(cb * SCH, SCH)], ss, sem)
            pltpu.async_copy(dst_hbm.at[pl.ds(cb * SCH, SCH)], dd, sem)

        def drain(b):
            ss, dd, sem = bufs[b]
            pltpu.make_async_copy(src_hbm.at[pl.ds(0, SCH)], ss, sem).wait()
            pltpu.make_async_copy(src_hbm.at[pl.ds(0, SCH)], dd, sem).wait()

        issue(0, 0)
        carry = (jnp.int32(0), jnp.int32(0))
        for cb in range(nb):
            b = cb % 2
            drain(b)
            if cb + 1 < nb:
                issue(cb + 1, 1 - b)
            ss, dd, _ = bufs[b]
            carry = lax.fori_loop(0, SCH // 16, make_append(ss, dd), carry)
        pos, flushed = carry
        n_total = flushed + pos

        zeros16 = jnp.zeros((16,), jnp.int32)
        pad16 = jnp.full((16,), NPW, jnp.int32)
        for j in range(G // 16):
            buf_s[pl.ds(pos + j * 16, 16)] = zeros16
            buf_l[pl.ds(pos + j * 16, 16)] = pad16
        pos = pos + G
        pos, flushed = lax.cond(pos >= K_FLUSH, do_flush, no_flush,
                                pos, flushed)

        off = pl.multiple_of(w * ECAP + flushed, 8)
        pltpu.sync_copy(buf_s.at[pl.ds(0, K_FLUSH)],
                        bsrc_hbm.at[pl.ds(off, K_FLUSH)])
        pltpu.sync_copy(buf_l.at[pl.ds(0, K_FLUSH)],
                        bloc_hbm.at[pl.ds(off, K_FLUSH)])

        cnt_v[pl.ds(0, 16)] = jnp.full((16,), n_total, jnp.int32)
        pltpu.sync_copy(cnt_v, cnt_hbm.at[pl.ds(pl.multiple_of(w * 16, 8), 16)])

        n = n_total
        ones_v = jnp.full((16,), 1, jnp.int32)
        lanes = lax.broadcasted_iota(jnp.int32, (16,), 0)
        onehots = [lanes == jnp.int32(t) for t in range(16)]

        for k in range(OFFW // 16):
            cnts_v[pl.ds(k * 16, 16)] = zero_v

        nbb = (n + (SCH - 1)) // SCH

        def bcount(bi, _):
            boff = pl.multiple_of(w * ECAP + bi * SCH, 8)
            pltpu.sync_copy(bloc_hbm.at[pl.ds(boff, SCH)], stage_da)
            nv = jnp.minimum(n - bi * SCH, SCH)
            ngrp = (nv + 15) // 16

            def grp(i2, _):
                lv = stage_da[pl.ds(i2 * 16, 16)]
                for t in range(16):
                    plsc.addupdate_scatter(cnts_v, [lv], ones_v,
                                           mask=onehots[t])
                return 0
            lax.fori_loop(0, ngrp, grp, 0)
            return 0
        lax.fori_loop(0, nbb, bcount, 0)

        carry = jnp.int32(0)
        for k in range(OFFW // 16):
            v = cnts_v[pl.ds(k * 16, 16)]
            cs = plsc.cumsum(v)
            offs_v[pl.ds(k * 16, 16)] = jnp.full((16,), carry, jnp.int32) + cs - v
            carry = carry + cs[15]
        pltpu.sync_copy(offs_v.at[pl.ds(0, OFFW)],
                        offs_hbm.at[pl.ds(pl.multiple_of(w * OFFW, 8), OFFW)])

        nrounds = (n + (CAPV - 1)) // CAPV

        def round_body(rd, _):
            win_lo = rd * CAPV
            win_lo_v = jnp.full((16,), win_lo, jnp.int32)

            def zbody(k2, _):
                sort_v[pl.ds(k2 * 16, 16)] = zero_v
                return 0
            lax.fori_loop(0, CAPV // 16, zbody, 0)
            for k in range(OFFW // 16):
                offs2_v[pl.ds(k * 16, 16)] = offs_v[pl.ds(k * 16, 16)]

            def cblk(bi, _):
                boff = pl.multiple_of(w * ECAP + bi * SCH, 8)
                pltpu.sync_copy(bloc_hbm.at[pl.ds(boff, SCH)], stage_da)
                pltpu.sync_copy(bsrc_hbm.at[pl.ds(boff, SCH)], stage_sa)
                nv = jnp.minimum(n - bi * SCH, SCH)
                ngrp = (nv + 15) // 16

                def grp(i2, _):
                    lv = stage_da[pl.ds(i2 * 16, 16)]
                    sv = stage_sa[pl.ds(i2 * 16, 16)]
                    base_e = bi * SCH + i2 * 16
                    for t in range(16):
                        ov = plsc.load_gather(offs2_v, [lv])
                        o = ov[t]
                        dst_i = jnp.clip(o - win_lo, 0, CAPV - 1)
                        valid = ((base_e + t < n) & (o >= win_lo)
                                 & (o < win_lo + CAPV))
                        mvec = onehots[t] & jnp.full((16,), valid, jnp.bool_)
                        plsc.store_scatter(sort_v,
                                           [jnp.full((16,), dst_i, jnp.int32)],
                                           sv, mask=mvec)
                        plsc.addupdate_scatter(offs2_v, [lv], ones_v,
                                               mask=onehots[t])
                    return 0
                lax.fori_loop(0, ngrp, grp, 0)
                return 0
            lax.fori_loop(0, nbb, cblk, 0)

            pltpu.sync_copy(
                sort_v.at[pl.ds(0, CAPV)],
                bsrt_hbm.at[pl.ds(pl.multiple_of(w * ECAP + win_lo, 8),
                                  CAPV)])
            return 0
        lax.fori_loop(0, nrounds, round_body, 0)

        n_rounded = ((n + 15) // 16) * 16

        def zbody2(k2, _):
            sort_v[pl.ds(k2 * 16, 16)] = zero_v
            return 0
        lax.fori_loop(0, G // 16, zbody2, 0)
        pltpu.sync_copy(
            sort_v.at[pl.ds(0, G)],
            bsrt_hbm.at[pl.ds(pl.multiple_of(w * ECAP + n_rounded, 8), G)])

    return bin_edges


IB = 4096


@functools.lru_cache(maxsize=None)
def _get_segmax(C):
    g = 64 if C > 128 else 128
    cpb = IB // g
    nj = C // 16

    @functools.partial(
        pl.kernel,
        out_type=jax.ShapeDtypeStruct((NPAD, C), jnp.float32),
        mesh=_sc_mesh(),
        scratch_types=[
            pltpu.VMEM((NPW, C), jnp.float32),
            pltpu.VMEM((g, C), jnp.float32),
            pltpu.VMEM((g, C), jnp.float32),
            pltpu.VMEM((IB,), jnp.int32),
            pltpu.VMEM((OFFW + 16,), jnp.int32),
            pltpu.VMEM((16,), jnp.int32),
            pltpu.SemaphoreType.DMA,
            pltpu.SemaphoreType.DMA,
        ],
        compiler_params=pltpu.CompilerParams(needs_layout_passes=False),
    )
    def seg_kernel(q_hbm, bsrt_hbm, offs_hbm, cnt_hbm, s_hbm,
                   acc, rows_a, rows_b, ibuf_s, offs_v, cnt_v, sem_a, sem_b):
        w = _worker_id()
        lo = w * NPW
        pltpu.sync_copy(cnt_hbm.at[pl.ds(pl.multiple_of(w * 16, 8), 16)],
                        cnt_v)
        n = cnt_v[pl.ds(0, 16)][0]
        pltpu.sync_copy(offs_hbm.at[pl.ds(pl.multiple_of(w * OFFW, 8), OFFW)],
                        offs_v.at[pl.ds(0, OFFW)])
        nchunks = (n + (g - 1)) // g
        nblocks = (nchunks + (cpb - 1)) // cpb

        neg = jnp.full((16,), _NEG_INF, jnp.float32)

        def init_body(i, _):
            for j in range(nj):
                acc[i, pl.ds(j * 16, 16)] = neg
            return 0
        lax.fori_loop(0, NPW, init_body, 0)

        def compute(rows, gbase, r_in):
            g_valid = jnp.minimum(n - gbase, g)

            def cond(st):
                return st[0] < g_valid

            def body(st):
                e, r = st
                seg_end = offs_v[pl.ds(r + 1, 16)][0] - gbase
                le = jnp.minimum(seg_end, g_valid)
                regs = tuple(acc[r, pl.ds(j * 16, 16)] for j in range(nj))

                def ebody(ei, rg):
                    return tuple(
                        jnp.maximum(rg[j], rows[ei, pl.ds(j * 16, 16)])
                        for j in range(nj))
                regs = lax.fori_loop(e, le, ebody, regs)
                for j in range(nj):
                    acc[r, pl.ds(j * 16, 16)] = regs[j]
                rn = jnp.where(le < g_valid, r + 1, r)
                return (le, rn)

            e_r = lax.while_loop(cond, body, (jnp.int32(0), r_in))
            return e_r[1]

        def gather(c, rows, sem):
            pltpu.async_copy(q_hbm.at[ibuf_s.at[pl.ds(c * g, g)]], rows, sem)

        def wait(rows, sem):
            pltpu.make_async_copy(q_hbm.at[ibuf_s.at[pl.ds(0, g)]],
                                  rows, sem).wait()

        def block_body(ib, r_in):
            boff = pl.multiple_of(w * ECAP + ib * IB, 8)
            pltpu.sync_copy(bsrt_hbm.at[pl.ds(boff, IB)], ibuf_s)
            ch = jnp.minimum(nchunks - ib * cpb, cpb)
            gather(0, rows_a, sem_a)
            blk_base = ib * IB

            def pair_body(p, r_c):
                c0 = 2 * p
                wait(rows_a, sem_a)

                @pl.when(c0 + 1 < ch)
                def _():
                    gather(c0 + 1, rows_b, sem_b)
                r_c = compute(rows_a, blk_base + c0 * g, r_c)

                def odd(r_c2):
                    wait(rows_b, sem_b)

                    @pl.when(c0 + 2 < ch)
                    def _():
                        gather(c0 + 2, rows_a, sem_a)
                    return compute(rows_b, blk_base + (c0 + 1) * g, r_c2)

                return lax.cond(c0 + 1 < ch, odd, lambda r_c2: r_c2, r_c)
            return lax.fori_loop(0, (ch + 1) // 2, pair_body, r_in)
        lax.fori_loop(0, nblocks, block_body, jnp.int32(0))

        pltpu.sync_copy(acc.at[pl.ds(0, NPW)],
                        s_hbm.at[pl.ds(pl.multiple_of(lo, 8), NPW)])

    return seg_kernel


_TR = 1000


def _tc_first(x, A, bias, C, QW):
    cin = x.shape[1]

    def body(x_ref, a_ref, b_ref, p_ref, q_ref):
        r = jnp.dot(x_ref[...], a_ref[...],
                    preferred_element_type=jnp.float32) + b_ref[...]
        p_ref[...] = r[:, :C]
        q = r[:, C:]
        if QW > C:
            q = jnp.concatenate(
                [q, jnp.zeros((q.shape[0], QW - C), jnp.float32)], axis=1)
        q_ref[...] = q

    return pl.pallas_call(
        body,
        grid=(N_NODES // _TR,),
        in_specs=[
            pl.BlockSpec((_TR, cin), lambda i: (i, 0)),
            pl.BlockSpec((cin, 2 * C), lambda i: (0, 0)),
            pl.BlockSpec((1, 2 * C), lambda i: (0, 0)),
        ],
        out_specs=[
            pl.BlockSpec((_TR, C), lambda i: (i, 0)),
            pl.BlockSpec((_TR, QW), lambda i: (i, 0)),
        ],
        out_shape=[jax.ShapeDtypeStruct((N_NODES, C), jnp.float32),
                   jax.ShapeDtypeStruct((N_NODES, QW), jnp.float32)],
    )(x, A, bias)


def _tc_mid(p_prev, s_prev, A, bias, C):
    cin = p_prev.shape[1]

    def body(p_ref, s_ref, a_ref, b_ref, po_ref, qo_ref):
        xv = jnp.maximum(p_ref[...] + s_ref[...], 0.0)
        r = jnp.dot(xv, a_ref[...],
                    preferred_element_type=jnp.float32) + b_ref[...]
        po_ref[...] = r[:, :C]
        qo_ref[...] = r[:, C:]

    return pl.pallas_call(
        body,
        grid=(N_NODES // _TR,),
        in_specs=[
            pl.BlockSpec((_TR, cin), lambda i: (i, 0)),
            pl.BlockSpec((_TR, cin), lambda i: (i, 0)),
            pl.BlockSpec((cin, 2 * C), lambda i: (0, 0)),
            pl.BlockSpec((1, 2 * C), lambda i: (0, 0)),
        ],
        out_specs=[
            pl.BlockSpec((_TR, C), lambda i: (i, 0)),
            pl.BlockSpec((_TR, C), lambda i: (i, 0)),
        ],
        out_shape=[jax.ShapeDtypeStruct((N_NODES, C), jnp.float32)] * 2,
    )(p_prev, s_prev, A, bias)


def _tc_final(p3, s3a, s3b, x0, W4, b4, W5, b5):
    def body(p_ref, sa_ref, sb_ref, x0_ref, w4_ref, b4_ref, w5_ref, b5_ref,
             o_ref):
        s = jnp.concatenate([sa_ref[...], sb_ref[...]], axis=1)
        xv = jnp.maximum(p_ref[...] + s, 0.0)
        h = jnp.maximum(
            jnp.dot(xv, w4_ref[...], preferred_element_type=jnp.float32)
            + b4_ref[...], 0.0)
        o_ref[...] = (jnp.dot(h, w5_ref[...],
                              preferred_element_type=jnp.float32)
                      + b5_ref[...] + x0_ref[...])

    return pl.pallas_call(
        body,
        grid=(N_NODES // _TR,),
        in_specs=[
            pl.BlockSpec((_TR, 512), lambda i: (i, 0)),
            pl.BlockSpec((_TR, 256), lambda i: (i, 0)),
            pl.BlockSpec((_TR, 256), lambda i: (i, 0)),
            pl.BlockSpec((_TR, 3), lambda i: (i, 0)),
            pl.BlockSpec((512, 256), lambda i: (0, 0)),
            pl.BlockSpec((1, 256), lambda i: (0, 0)),
            pl.BlockSpec((256, 3), lambda i: (0, 0)),
            pl.BlockSpec((1, 3), lambda i: (0, 0)),
        ],
        out_specs=pl.BlockSpec((_TR, 3), lambda i: (i, 0)),
        out_shape=jax.ShapeDtypeStruct((N_NODES, 3), jnp.float32),
    )(p3, s3a, s3b, x0, W4, b4, W5, b5)


def _split_weights(W, b, cin):
    wa, wb = W[:cin], W[cin:]
    A = jnp.concatenate([wa - wb, wb], axis=1)
    bias = jnp.concatenate([b, jnp.zeros_like(b)])[None, :]
    return A, bias


def kernel(x, edge_index, W1, b1, W2, b2, W3, b3, W4, b4, W5, b5):
    src = edge_index[0]
    dst = edge_index[1]

    bsrc, bloc, counts, bsrt, offs = _get_bin_kernel()(src, dst)

    A1, bias1 = _split_weights(W1, b1, 3)
    A2, bias2 = _split_weights(W2, b2, 64)
    A3, bias3 = _split_weights(W3, b3, 128)

    P1, Q1 = _tc_first(x, A1, bias1, 64, 128)
    S1 = _get_segmax(128)(Q1, bsrt, offs, counts)[:N_NODES, :64]

    P2, Q2 = _tc_mid(P1, S1, A2, bias2, 128)
    S2 = _get_segmax(128)(Q2, bsrt, offs, counts)[:N_NODES]

    P3, Q3 = _tc_mid(P2, S2, A3, bias3, 512)
    S3a = _get_segmax(256)(Q3[:, :256], bsrt, offs, counts)[:N_NODES]
    S3b = _get_segmax(256)(Q3[:, 256:], bsrt, offs, counts)[:N_NODES]

    return _tc_final(P3, S3a, S3b, x, W4, b4[None, :], W5, b5[None, :])


# speedup vs baseline: 2.0565x; 1.1623x over previous
import functools

import jax
import jax.numpy as jnp
from jax import lax
from jax.experimental import pallas as pl
from jax.experimental.pallas import tpu as pltpu
from jax.experimental.pallas import tpu_sc as plsc

N_NODES = 10000
N_EDGES = 160000

NC = 2
NS = 16
NW = NC * NS
NPW = 320
NPAD = NW * NPW

K_FLUSH = 4096
G = 128
ECAP = N_EDGES + K_FLUSH + 256
SCH = 8000
BUFCAP = K_FLUSH + 192
CAPV = 16384
OFFW = 352

_NEG_INF = float("-inf")


def _worker_id():
    return lax.axis_index("s") * NC + lax.axis_index("c")


def _sc_mesh():
    return plsc.VectorSubcoreMesh(
        core_axis_name="c", subcore_axis_name="s",
        num_cores=NC, num_subcores=NS)


@functools.lru_cache(maxsize=None)
def _get_bin_kernel():
    @functools.partial(
        pl.kernel,
        out_type=[
            jax.ShapeDtypeStruct((NW * ECAP,), jnp.int32),
            jax.ShapeDtypeStruct((NW * ECAP,), jnp.int32),
            jax.ShapeDtypeStruct((NW * 16,), jnp.int32),
            jax.ShapeDtypeStruct((NW * ECAP,), jnp.int32),
            jax.ShapeDtypeStruct((NW * OFFW,), jnp.int32),
        ],
        mesh=_sc_mesh(),
        scratch_types=[
            pltpu.VMEM((SCH,), jnp.int32),
            pltpu.VMEM((SCH,), jnp.int32),
            pltpu.VMEM((SCH,), jnp.int32),
            pltpu.VMEM((SCH,), jnp.int32),
            pltpu.VMEM((BUFCAP,), jnp.int32),
            pltpu.VMEM((BUFCAP,), jnp.int32),
            pltpu.VMEM((16,), jnp.int32),
            pltpu.VMEM((OFFW,), jnp.int32),
            pltpu.VMEM((OFFW,), jnp.int32),
            pltpu.VMEM((OFFW,), jnp.int32),
            pltpu.VMEM((CAPV,), jnp.int32),
            pltpu.SemaphoreType.DMA,
            pltpu.SemaphoreType.DMA,
        ],
        compiler_params=pltpu.CompilerParams(needs_layout_passes=False),
    )
    def bin_edges(src_hbm, dst_hbm, bsrc_hbm, bloc_hbm, cnt_hbm,
                  bsrt_hbm, offs_hbm,
                  stage_sa, stage_da, stage_sb, stage_db,
                  buf_s, buf_l, cnt_v, cnts_v, offs_v, offs2_v, sort_v,
                  sem_a, sem_b):
        w = _worker_id()
        lo = w * NPW

        def do_flush(pos, flushed):
            off = pl.multiple_of(w * ECAP + flushed, 8)
            pltpu.sync_copy(buf_s.at[pl.ds(0, K_FLUSH)],
                            bsrc_hbm.at[pl.ds(off, K_FLUSH)])
            pltpu.sync_copy(buf_l.at[pl.ds(0, K_FLUSH)],
                            bloc_hbm.at[pl.ds(off, K_FLUSH)])
            ts = buf_s[pl.ds(K_FLUSH, 16)]
            tl = buf_l[pl.ds(K_FLUSH, 16)]
            buf_s[pl.ds(0, 16)] = ts
            buf_l[pl.ds(0, 16)] = tl
            return pos - K_FLUSH, flushed + K_FLUSH

        def no_flush(pos, flushed):
            return pos, flushed

        lo_v = jnp.full((16,), lo, jnp.int32)
        hi_v = jnp.full((16,), lo + NPW, jnp.int32)
        zero_v = jnp.zeros((16,), jnp.int32)
        one_v = jnp.full((16,), 1, jnp.int32)

        def make_append(ss, dd):
            def append_chunk(i, carry):
                pos, flushed = carry
                d0 = dd[pl.ds(i * 32, 16)]
                s0 = ss[pl.ds(i * 32, 16)]
                d1 = dd[pl.ds(i * 32 + 16, 16)]
                s1 = ss[pl.ds(i * 32 + 16, 16)]
                m0 = (d0 >= lo_v) & (d0 < hi_v)
                m1 = (d1 >= lo_v) & (d1 < hi_v)
                cs0 = plsc.cumsum(jnp.where(m0, one_v, zero_v))
                cs1 = plsc.cumsum(jnp.where(m1, one_v, zero_v))
                pos_v = jnp.full((16,), pos, jnp.int32)
                idx0 = jnp.maximum(pos_v + cs0 - one_v, zero_v)
                c0 = cs0[15]
                idx1 = jnp.maximum(pos_v + c0 + cs1 - one_v, zero_v)
                plsc.store_scatter(buf_s, [idx0], s0, mask=m0)
                plsc.store_scatter(buf_l, [idx0], d0 - lo_v, mask=m0)
                plsc.store_scatter(buf_s, [idx1], s1, mask=m1)
                plsc.store_scatter(buf_l, [idx1], d1 - lo_v, mask=m1)
                pos = pos + c0 + cs1[15]
                return lax.cond(pos >= K_FLUSH, do_flush, no_flush,
                                pos, flushed)
            return append_chunk

        nb = N_EDGES // SCH
        bufs = [(stage_sa, stage_da, sem_a), (stage_sb, stage_db, sem_b)]

        def issue(cb, b):
            ss, dd, sem = bufs[b]
            pltpu.async_copy(src_hbm.at[pl.ds(cb * SCH, SCH)], ss, sem)
            pltpu.async_copy(dst_hbm.at[pl.ds(cb * SCH, SCH)], dd, sem)

        def drain(b):
            ss, dd, sem = bufs[b]
            pltpu.make_async_copy(src_hbm.at[pl.ds(0, SCH)], ss, sem).wait()
            pltpu.make_async_copy(src_hbm.at[pl.ds(0, SCH)], dd, sem).wait()

        issue(0, 0)
        carry = (jnp.int32(0), jnp.int32(0))
        for cb in range(nb):
            b = cb % 2
            drain(b)
            if cb + 1 < nb:
                issue(cb + 1, 1 - b)
            ss, dd, _ = bufs[b]
            carry = lax.fori_loop(0, SCH // 32, make_append(ss, dd), carry)
        pos, flushed = carry
        n_total = flushed + pos

        zeros16 = jnp.zeros((16,), jnp.int32)
        pad16 = jnp.full((16,), NPW, jnp.int32)
        for j in range(G // 16):
            buf_s[pl.ds(pos + j * 16, 16)] = zeros16
            buf_l[pl.ds(pos + j * 16, 16)] = pad16
        pos = pos + G
        pos, flushed = lax.cond(pos >= K_FLUSH, do_flush, no_flush,
                                pos, flushed)

        off = pl.multiple_of(w * ECAP + flushed, 8)
        pltpu.sync_copy(buf_s.at[pl.ds(0, K_FLUSH)],
                        bsrc_hbm.at[pl.ds(off, K_FLUSH)])
        pltpu.sync_copy(buf_l.at[pl.ds(0, K_FLUSH)],
                        bloc_hbm.at[pl.ds(off, K_FLUSH)])

        cnt_v[pl.ds(0, 16)] = jnp.full((16,), n_total, jnp.int32)
        pltpu.sync_copy(cnt_v, cnt_hbm.at[pl.ds(pl.multiple_of(w * 16, 8), 16)])

        n = n_total
        ones_v = jnp.full((16,), 1, jnp.int32)
        lanes = lax.broadcasted_iota(jnp.int32, (16,), 0)
        onehots = [lanes == jnp.int32(t) for t in range(16)]

        for k in range(OFFW // 16):
            cnts_v[pl.ds(k * 16, 16)] = zero_v

        nbb = (n + (SCH - 1)) // SCH

        def bcount(bi, _):
            boff = pl.multiple_of(w * ECAP + bi * SCH, 8)
            pltpu.sync_copy(bloc_hbm.at[pl.ds(boff, SCH)], stage_da)
            nv = jnp.minimum(n - bi * SCH, SCH)
            ngrp = (nv + 15) // 16

            def grp(i2, _):
                lv = stage_da[pl.ds(i2 * 16, 16)]
                for t in range(16):
                    plsc.addupdate_scatter(cnts_v, [lv], ones_v,
                                           mask=onehots[t])
                return 0
            lax.fori_loop(0, ngrp, grp, 0)
            return 0
        lax.fori_loop(0, nbb, bcount, 0)

        carry = jnp.int32(0)
        for k in range(OFFW // 16):
            v = cnts_v[pl.ds(k * 16, 16)]
            cs = plsc.cumsum(v)
            offs_v[pl.ds(k * 16, 16)] = jnp.full((16,), carry, jnp.int32) + cs - v
            carry = carry + cs[15]
        pltpu.sync_copy(offs_v.at[pl.ds(0, OFFW)],
                        offs_hbm.at[pl.ds(pl.multiple_of(w * OFFW, 8), OFFW)])

        nrounds = (n + (CAPV - 1)) // CAPV

        def round_body(rd, _):
            win_lo = rd * CAPV
            win_lo_v = jnp.full((16,), win_lo, jnp.int32)

            def zbody(k2, _):
                sort_v[pl.ds(k2 * 16, 16)] = zero_v
                return 0
            lax.fori_loop(0, CAPV // 16, zbody, 0)
            for k in range(OFFW // 16):
                offs2_v[pl.ds(k * 16, 16)] = offs_v[pl.ds(k * 16, 16)]

            def cblk(bi, _):
                boff = pl.multiple_of(w * ECAP + bi * SCH, 8)
                pltpu.sync_copy(bloc_hbm.at[pl.ds(boff, SCH)], stage_da)
                pltpu.sync_copy(bsrc_hbm.at[pl.ds(boff, SCH)], stage_sa)
                nv = jnp.minimum(n - bi * SCH, SCH)
                ngrp = (nv + 15) // 16

                def grp(i2, _):
                    lv = stage_da[pl.ds(i2 * 16, 16)]
                    sv = stage_sa[pl.ds(i2 * 16, 16)]
                    base_e = bi * SCH + i2 * 16
                    for t in range(16):
                        ov = plsc.load_gather(offs2_v, [lv])
                        o = ov[t]
                        dst_i = jnp.clip(o - win_lo, 0, CAPV - 1)
                        valid = ((base_e + t < n) & (o >= win_lo)
                                 & (o < win_lo + CAPV))
                        mvec = onehots[t] & jnp.full((16,), valid, jnp.bool_)
                        plsc.store_scatter(sort_v,
                                           [jnp.full((16,), dst_i, jnp.int32)],
                                           sv, mask=mvec)
                        plsc.addupdate_scatter(offs2_v, [lv], ones_v,
                                               mask=onehots[t])
                    return 0
                lax.fori_loop(0, ngrp, grp, 0)
                return 0
            lax.fori_loop(0, nbb, cblk, 0)

            pltpu.sync_copy(
                sort_v.at[pl.ds(0, CAPV)],
                bsrt_hbm.at[pl.ds(pl.multiple_of(w * ECAP + win_lo, 8),
                                  CAPV)])
            return 0
        lax.fori_loop(0, nrounds, round_body, 0)

        n_rounded = ((n + 15) // 16) * 16

        def zbody2(k2, _):
            sort_v[pl.ds(k2 * 16, 16)] = zero_v
            return 0
        lax.fori_loop(0, G // 16, zbody2, 0)
        pltpu.sync_copy(
            sort_v.at[pl.ds(0, G)],
            bsrt_hbm.at[pl.ds(pl.multiple_of(w * ECAP + n_rounded, 8), G)])

    return bin_edges


IB = 8192


@functools.lru_cache(maxsize=None)
def _get_segmax(C):
    g = 64 if C > 128 else 128
    cpb = IB // g
    nj = C // 16

    @functools.partial(
        pl.kernel,
        out_type=jax.ShapeDtypeStruct((NPAD, C), jnp.float32),
        mesh=_sc_mesh(),
        scratch_types=[
            pltpu.VMEM((NPW, C), jnp.float32),
            pltpu.VMEM((g, C), jnp.float32),
            pltpu.VMEM((g, C), jnp.float32),
            pltpu.VMEM((IB,), jnp.int32),
            pltpu.VMEM((OFFW + 16,), jnp.int32),
            pltpu.VMEM((16,), jnp.int32),
            pltpu.SemaphoreType.DMA,
            pltpu.SemaphoreType.DMA,
        ],
        compiler_params=pltpu.CompilerParams(needs_layout_passes=False),
    )
    def seg_kernel(q_hbm, bsrt_hbm, offs_hbm, cnt_hbm, s_hbm,
                   acc, rows_a, rows_b, ibuf_s, offs_v, cnt_v, sem_a, sem_b):
        w = _worker_id()
        lo = w * NPW
        pltpu.sync_copy(cnt_hbm.at[pl.ds(pl.multiple_of(w * 16, 8), 16)],
                        cnt_v)
        n = cnt_v[pl.ds(0, 16)][0]
        pltpu.sync_copy(offs_hbm.at[pl.ds(pl.multiple_of(w * OFFW, 8), OFFW)],
                        offs_v.at[pl.ds(0, OFFW)])
        nchunks = (n + (g - 1)) // g
        nblocks = (nchunks + (cpb - 1)) // cpb

        neg = jnp.full((16,), _NEG_INF, jnp.float32)

        def init_body(i, _):
            for j in range(nj):
                acc[i, pl.ds(j * 16, 16)] = neg
            return 0
        lax.fori_loop(0, NPW, init_body, 0)

        def compute(rows, gbase, r_in):
            g_valid = jnp.minimum(n - gbase, g)

            def cond(st):
                return st[0] < g_valid

            def body(st):
                e, r = st
                seg_end = offs_v[pl.ds(r + 1, 16)][0] - gbase
                le = jnp.minimum(seg_end, g_valid)
                regs = tuple(acc[r, pl.ds(j * 16, 16)] for j in range(nj))

                def ebody(ei, rg):
                    return tuple(
                        jnp.maximum(rg[j], rows[ei, pl.ds(j * 16, 16)])
                        for j in range(nj))
                regs = lax.fori_loop(e, le, ebody, regs)
                for j in range(nj):
                    acc[r, pl.ds(j * 16, 16)] = regs[j]
                rn = jnp.where(le < g_valid, r + 1, r)
                return (le, rn)

            e_r = lax.while_loop(cond, body, (jnp.int32(0), r_in))
            return e_r[1]

        def gather(c, rows, sem):
            pltpu.async_copy(q_hbm.at[ibuf_s.at[pl.ds(c * g, g)]], rows, sem)

        def wait(rows, sem):
            pltpu.make_async_copy(q_hbm.at[ibuf_s.at[pl.ds(0, g)]],
                                  rows, sem).wait()

        def block_body(ib, r_in):
            boff = pl.multiple_of(w * ECAP + ib * IB, 8)
            pltpu.sync_copy(bsrt_hbm.at[pl.ds(boff, IB)], ibuf_s)
            ch = jnp.minimum(nchunks - ib * cpb, cpb)
            gather(0, rows_a, sem_a)
            blk_base = ib * IB

            def pair_body(p, r_c):
                c0 = 2 * p
                wait(rows_a, sem_a)

                @pl.when(c0 + 1 < ch)
                def _():
                    gather(c0 + 1, rows_b, sem_b)
                r_c = compute(rows_a, blk_base + c0 * g, r_c)

                def odd(r_c2):
                    wait(rows_b, sem_b)

                    @pl.when(c0 + 2 < ch)
                    def _():
                        gather(c0 + 2, rows_a, sem_a)
                    return compute(rows_b, blk_base + (c0 + 1) * g, r_c2)

                return lax.cond(c0 + 1 < ch, odd, lambda r_c2: r_c2, r_c)
            return lax.fori_loop(0, (ch + 1) // 2, pair_body, r_in)
        lax.fori_loop(0, nblocks, block_body, jnp.int32(0))

        pltpu.sync_copy(acc.at[pl.ds(0, NPW)],
                        s_hbm.at[pl.ds(pl.multiple_of(lo, 8), NPW)])

    return seg_kernel


_TR = 1000


def _tc_first(x, A, bias, C, QW):
    cin = x.shape[1]

    def body(x_ref, a_ref, b_ref, p_ref, q_ref):
        r = jnp.dot(x_ref[...], a_ref[...],
                    preferred_element_type=jnp.float32) + b_ref[...]
        p_ref[...] = r[:, :C]
        q = r[:, C:]
        if QW > C:
            q = jnp.concatenate(
                [q, jnp.zeros((q.shape[0], QW - C), jnp.float32)], axis=1)
        q_ref[...] = q

    return pl.pallas_call(
        body,
        grid=(N_NODES // _TR,),
        in_specs=[
            pl.BlockSpec((_TR, cin), lambda i: (i, 0)),
            pl.BlockSpec((cin, 2 * C), lambda i: (0, 0)),
            pl.BlockSpec((1, 2 * C), lambda i: (0, 0)),
        ],
        out_specs=[
            pl.BlockSpec((_TR, C), lambda i: (i, 0)),
            pl.BlockSpec((_TR, QW), lambda i: (i, 0)),
        ],
        out_shape=[jax.ShapeDtypeStruct((N_NODES, C), jnp.float32),
                   jax.ShapeDtypeStruct((N_NODES, QW), jnp.float32)],
    )(x, A, bias)


def _tc_mid(p_prev, s_prev, A, bias, C, split_q=False):
    cin = p_prev.shape[1]
    nq = 2 if split_q else 1
    qw = C // nq

    def body(p_ref, s_ref, a_ref, b_ref, po_ref, *q_refs):
        xv = jnp.maximum(p_ref[...] + s_ref[...], 0.0)
        r = jnp.dot(xv, a_ref[...],
                    preferred_element_type=jnp.float32) + b_ref[...]
        po_ref[...] = r[:, :C]
        for k, q_ref in enumerate(q_refs):
            q_ref[...] = r[:, C + k * qw:C + (k + 1) * qw]

    return pl.pallas_call(
        body,
        grid=(N_NODES // _TR,),
        in_specs=[
            pl.BlockSpec((_TR, cin), lambda i: (i, 0)),
            pl.BlockSpec((_TR, cin), lambda i: (i, 0)),
            pl.BlockSpec((cin, 2 * C), lambda i: (0, 0)),
            pl.BlockSpec((1, 2 * C), lambda i: (0, 0)),
        ],
        out_specs=[pl.BlockSpec((_TR, C), lambda i: (i, 0))]
        + [pl.BlockSpec((_TR, qw), lambda i: (i, 0)) for _ in range(nq)],
        out_shape=[jax.ShapeDtypeStruct((N_NODES, C), jnp.float32)]
        + [jax.ShapeDtypeStruct((N_NODES, qw), jnp.float32)
           for _ in range(nq)],
    )(p_prev, s_prev, A, bias)


def _tc_final(p3, s3a, s3b, x0, W4, b4, W5, b5):
    def body(p_ref, sa_ref, sb_ref, x0_ref, w4_ref, b4_ref, w5_ref, b5_ref,
             o_ref):
        s = jnp.concatenate([sa_ref[...], sb_ref[...]], axis=1)
        xv = jnp.maximum(p_ref[...] + s, 0.0)
        h = jnp.maximum(
            jnp.dot(xv, w4_ref[...], preferred_element_type=jnp.float32)
            + b4_ref[...], 0.0)
        o_ref[...] = (jnp.dot(h, w5_ref[...],
                              preferred_element_type=jnp.float32)
                      + b5_ref[...] + x0_ref[...])

    return pl.pallas_call(
        body,
        grid=(N_NODES // _TR,),
        in_specs=[
            pl.BlockSpec((_TR, 512), lambda i: (i, 0)),
            pl.BlockSpec((_TR, 256), lambda i: (i, 0)),
            pl.BlockSpec((_TR, 256), lambda i: (i, 0)),
            pl.BlockSpec((_TR, 3), lambda i: (i, 0)),
            pl.BlockSpec((512, 256), lambda i: (0, 0)),
            pl.BlockSpec((1, 256), lambda i: (0, 0)),
            pl.BlockSpec((256, 3), lambda i: (0, 0)),
            pl.BlockSpec((1, 3), lambda i: (0, 0)),
        ],
        out_specs=pl.BlockSpec((_TR, 3), lambda i: (i, 0)),
        out_shape=jax.ShapeDtypeStruct((N_NODES, 3), jnp.float32),
    )(p3, s3a, s3b, x0, W4, b4, W5, b5)


def _split_weights(W, b, cin):
    wa, wb = W[:cin], W[cin:]
    A = jnp.concatenate([wa - wb, wb], axis=1)
    bias = jnp.concatenate([b, jnp.zeros_like(b)])[None, :]
    return A, bias


def kernel(x, edge_index, W1, b1, W2, b2, W3, b3, W4, b4, W5, b5):
    src = edge_index[0]
    dst = edge_index[1]

    bsrc, bloc, counts, bsrt, offs = _get_bin_kernel()(src, dst)

    A1, bias1 = _split_weights(W1, b1, 3)
    A2, bias2 = _split_weights(W2, b2, 64)
    A3, bias3 = _split_weights(W3, b3, 128)

    P1, Q1 = _tc_first(x, A1, bias1, 64, 128)
    S1 = _get_segmax(128)(Q1, bsrt, offs, counts)[:N_NODES, :64]

    P2, Q2 = _tc_mid(P1, S1, A2, bias2, 128)
    S2 = _get_segmax(128)(Q2, bsrt, offs, counts)[:N_NODES]

    P3, Q3a, Q3b = _tc_mid(P2, S2, A3, bias3, 512, split_q=True)
    S3a = _get_segmax(256)(Q3a, bsrt, offs, counts)[:N_NODES]
    S3b = _get_segmax(256)(Q3b, bsrt, offs, counts)[:N_NODES]

    return _tc_final(P3, S3a, S3b, x, W4, b4[None, :], W5, b5[None, :])


# speedup vs baseline: 2.3127x; 1.1246x over previous
import functools

import jax
import jax.numpy as jnp
from jax import lax
from jax.experimental import pallas as pl
from jax.experimental.pallas import tpu as pltpu
from jax.experimental.pallas import tpu_sc as plsc

N_NODES = 10000
N_EDGES = 160000

NC = 2
NS = 16
NW = NC * NS
NPW = 320
NPAD = NW * NPW

K_FLUSH = 4096
G = 128
ECAP = N_EDGES + K_FLUSH + 256
SCH = 8000
BUFCAP = K_FLUSH + 192
CAPV = 16384
OFFW = 352

_NEG_INF = float("-inf")


def _worker_id():
    return lax.axis_index("s") * NC + lax.axis_index("c")


def _sc_mesh():
    return plsc.VectorSubcoreMesh(
        core_axis_name="c", subcore_axis_name="s",
        num_cores=NC, num_subcores=NS)


@functools.lru_cache(maxsize=None)
def _get_bin_kernel():
    @functools.partial(
        pl.kernel,
        out_type=[
            jax.ShapeDtypeStruct((NW * ECAP,), jnp.int32),
            jax.ShapeDtypeStruct((NW * ECAP,), jnp.int32),
            jax.ShapeDtypeStruct((NW * 16,), jnp.int32),
            jax.ShapeDtypeStruct((NW * ECAP,), jnp.int32),
            jax.ShapeDtypeStruct((NW * OFFW,), jnp.int32),
        ],
        mesh=_sc_mesh(),
        scratch_types=[
            pltpu.VMEM((SCH,), jnp.int32),
            pltpu.VMEM((SCH,), jnp.int32),
            pltpu.VMEM((SCH,), jnp.int32),
            pltpu.VMEM((SCH,), jnp.int32),
            pltpu.VMEM((BUFCAP,), jnp.int32),
            pltpu.VMEM((BUFCAP,), jnp.int32),
            pltpu.VMEM((16,), jnp.int32),
            pltpu.VMEM((OFFW,), jnp.int32),
            pltpu.VMEM((OFFW,), jnp.int32),
            pltpu.VMEM((OFFW,), jnp.int32),
            pltpu.VMEM((CAPV,), jnp.int32),
            pltpu.SemaphoreType.DMA,
            pltpu.SemaphoreType.DMA,
        ],
        compiler_params=pltpu.CompilerParams(needs_layout_passes=False),
    )
    def bin_edges(src_hbm, dst_hbm, bsrc_hbm, bloc_hbm, cnt_hbm,
                  bsrt_hbm, offs_hbm,
                  stage_sa, stage_da, stage_sb, stage_db,
                  buf_s, buf_l, cnt_v, cnts_v, offs_v, offs2_v, sort_v,
                  sem_a, sem_b):
        w = _worker_id()
        lo = w * NPW

        def do_flush(pos, flushed):
            off = pl.multiple_of(w * ECAP + flushed, 8)
            pltpu.sync_copy(buf_s.at[pl.ds(0, K_FLUSH)],
                            bsrc_hbm.at[pl.ds(off, K_FLUSH)])
            pltpu.sync_copy(buf_l.at[pl.ds(0, K_FLUSH)],
                            bloc_hbm.at[pl.ds(off, K_FLUSH)])
            ts = buf_s[pl.ds(K_FLUSH, 16)]
            tl = buf_l[pl.ds(K_FLUSH, 16)]
            buf_s[pl.ds(0, 16)] = ts
            buf_l[pl.ds(0, 16)] = tl
            return pos - K_FLUSH, flushed + K_FLUSH

        def no_flush(pos, flushed):
            return pos, flushed

        lo_v = jnp.full((16,), lo, jnp.int32)
        hi_v = jnp.full((16,), lo + NPW, jnp.int32)
        zero_v = jnp.zeros((16,), jnp.int32)
        one_v = jnp.full((16,), 1, jnp.int32)

        def make_append(ss, dd):
            def append_chunk(i, carry):
                pos, flushed = carry
                d0 = dd[pl.ds(i * 32, 16)]
                s0 = ss[pl.ds(i * 32, 16)]
                d1 = dd[pl.ds(i * 32 + 16, 16)]
                s1 = ss[pl.ds(i * 32 + 16, 16)]
                m0 = (d0 >= lo_v) & (d0 < hi_v)
                m1 = (d1 >= lo_v) & (d1 < hi_v)
                cs0 = plsc.cumsum(jnp.where(m0, one_v, zero_v))
                cs1 = plsc.cumsum(jnp.where(m1, one_v, zero_v))
                pos_v = jnp.full((16,), pos, jnp.int32)
                idx0 = jnp.maximum(pos_v + cs0 - one_v, zero_v)
                c0 = cs0[15]
                idx1 = jnp.maximum(pos_v + c0 + cs1 - one_v, zero_v)
                plsc.store_scatter(buf_s, [idx0], s0, mask=m0)
                plsc.store_scatter(buf_l, [idx0], d0 - lo_v, mask=m0)
                plsc.store_scatter(buf_s, [idx1], s1, mask=m1)
                plsc.store_scatter(buf_l, [idx1], d1 - lo_v, mask=m1)
                pos = pos + c0 + cs1[15]
                return lax.cond(pos >= K_FLUSH, do_flush, no_flush,
                                pos, flushed)
            return append_chunk

        nb = N_EDGES // SCH
        bufs = [(stage_sa, stage_da, sem_a), (stage_sb, stage_db, sem_b)]

        def issue(cb, b):
            ss, dd, sem = bufs[b]
            pltpu.async_copy(src_hbm.at[pl.ds(cb * SCH, SCH)], ss, sem)
            pltpu.async_copy(dst_hbm.at[pl.ds(cb * SCH, SCH)], dd, sem)

        def drain(b):
            ss, dd, sem = bufs[b]
            pltpu.make_async_copy(src_hbm.at[pl.ds(0, SCH)], ss, sem).wait()
            pltpu.make_async_copy(src_hbm.at[pl.ds(0, SCH)], dd, sem).wait()

        issue(0, 0)
        carry = (jnp.int32(0), jnp.int32(0))
        for cb in range(nb):
            b = cb % 2
            drain(b)
            if cb + 1 < nb:
                issue(cb + 1, 1 - b)
            ss, dd, _ = bufs[b]
            carry = lax.fori_loop(0, SCH // 32, make_append(ss, dd), carry)
        pos, flushed = carry
        n_total = flushed + pos

        zeros16 = jnp.zeros((16,), jnp.int32)
        pad16 = jnp.full((16,), NPW, jnp.int32)
        for j in range(G // 16):
            buf_s[pl.ds(pos + j * 16, 16)] = zeros16
            buf_l[pl.ds(pos + j * 16, 16)] = pad16
        pos = pos + G
        pos, flushed = lax.cond(pos >= K_FLUSH, do_flush, no_flush,
                                pos, flushed)

        off = pl.multiple_of(w * ECAP + flushed, 8)
        pltpu.sync_copy(buf_s.at[pl.ds(0, K_FLUSH)],
                        bsrc_hbm.at[pl.ds(off, K_FLUSH)])
        pltpu.sync_copy(buf_l.at[pl.ds(0, K_FLUSH)],
                        bloc_hbm.at[pl.ds(off, K_FLUSH)])

        cnt_v[pl.ds(0, 16)] = jnp.full((16,), n_total, jnp.int32)
        pltpu.sync_copy(cnt_v, cnt_hbm.at[pl.ds(pl.multiple_of(w * 16, 8), 16)])

        n = n_total
        ones_v = jnp.full((16,), 1, jnp.int32)
        lanes = lax.broadcasted_iota(jnp.int32, (16,), 0)
        onehots = [lanes == jnp.int32(t) for t in range(16)]

        for k in range(OFFW // 16):
            cnts_v[pl.ds(k * 16, 16)] = zero_v

        nbb = (n + (SCH - 1)) // SCH

        def bcount(bi, _):
            boff = pl.multiple_of(w * ECAP + bi * SCH, 8)
            pltpu.sync_copy(bloc_hbm.at[pl.ds(boff, SCH)], stage_da)
            nv = jnp.minimum(n - bi * SCH, SCH)
            ngrp = (nv + 15) // 16

            def grp(i2, _):
                lv = stage_da[pl.ds(i2 * 16, 16)]
                for t in range(16):
                    plsc.addupdate_scatter(cnts_v, [lv], ones_v,
                                           mask=onehots[t])
                return 0
            lax.fori_loop(0, ngrp, grp, 0)
            return 0
        lax.fori_loop(0, nbb, bcount, 0)

        carry = jnp.int32(0)
        for k in range(OFFW // 16):
            v = cnts_v[pl.ds(k * 16, 16)]
            cs = plsc.cumsum(v)
            offs_v[pl.ds(k * 16, 16)] = jnp.full((16,), carry, jnp.int32) + cs - v
            carry = carry + cs[15]
        pltpu.sync_copy(offs_v.at[pl.ds(0, OFFW)],
                        offs_hbm.at[pl.ds(pl.multiple_of(w * OFFW, 8), OFFW)])

        nrounds = (n + (CAPV - 1)) // CAPV

        def round_body(rd, _):
            win_lo = rd * CAPV
            win_lo_v = jnp.full((16,), win_lo, jnp.int32)

            def zbody(k2, _):
                sort_v[pl.ds(k2 * 16, 16)] = zero_v
                return 0
            lax.fori_loop(0, CAPV // 16, zbody, 0)
            for k in range(OFFW // 16):
                offs2_v[pl.ds(k * 16, 16)] = offs_v[pl.ds(k * 16, 16)]

            def cblk(bi, _):
                boff = pl.multiple_of(w * ECAP + bi * SCH, 8)
                pltpu.sync_copy(bloc_hbm.at[pl.ds(boff, SCH)], stage_da)
                pltpu.sync_copy(bsrc_hbm.at[pl.ds(boff, SCH)], stage_sa)
                nv = jnp.minimum(n - bi * SCH, SCH)
                ngrp = (nv + 15) // 16

                def grp(i2, _):
                    lv = stage_da[pl.ds(i2 * 16, 16)]
                    sv = stage_sa[pl.ds(i2 * 16, 16)]
                    base_e = bi * SCH + i2 * 16
                    for t in range(16):
                        ov = plsc.load_gather(offs2_v, [lv])
                        o = ov[t]
                        dst_i = jnp.clip(o - win_lo, 0, CAPV - 1)
                        valid = ((base_e + t < n) & (o >= win_lo)
                                 & (o < win_lo + CAPV))
                        mvec = onehots[t] & jnp.full((16,), valid, jnp.bool_)
                        plsc.store_scatter(sort_v,
                                           [jnp.full((16,), dst_i, jnp.int32)],
                                           sv, mask=mvec)
                        plsc.addupdate_scatter(offs2_v, [lv], ones_v,
                                               mask=onehots[t])
                    return 0
                lax.fori_loop(0, ngrp, grp, 0)
                return 0
            lax.fori_loop(0, nbb, cblk, 0)

            pltpu.sync_copy(
                sort_v.at[pl.ds(0, CAPV)],
                bsrt_hbm.at[pl.ds(pl.multiple_of(w * ECAP + win_lo, 8),
                                  CAPV)])
            return 0
        lax.fori_loop(0, nrounds, round_body, 0)

        n_rounded = ((n + 15) // 16) * 16

        def zbody2(k2, _):
            sort_v[pl.ds(k2 * 16, 16)] = zero_v
            return 0
        lax.fori_loop(0, G // 16, zbody2, 0)
        pltpu.sync_copy(
            sort_v.at[pl.ds(0, G)],
            bsrt_hbm.at[pl.ds(pl.multiple_of(w * ECAP + n_rounded, 8), G)])

    return bin_edges


IB = 8192


@functools.lru_cache(maxsize=None)
def _get_segmax(C, QW=None):
    QW = QW or C
    if C > 128:
        g, nbuf, ib = 48, 3, 7680
    else:
        g, nbuf, ib = 128, 4, 8192
    cpb = ib // g
    nj = C // 16

    @functools.partial(
        pl.kernel,
        out_type=jax.ShapeDtypeStruct((NPAD, C), jnp.float32),
        mesh=_sc_mesh(),
        scratch_types=(
            [pltpu.VMEM((NPW, C), jnp.float32)]
            + [pltpu.VMEM((g, QW), jnp.float32) for _ in range(nbuf)]
            + [
                pltpu.VMEM((ib,), jnp.int32),
                pltpu.VMEM((OFFW + 16,), jnp.int32),
                pltpu.VMEM((16,), jnp.int32),
            ]
            + [pltpu.SemaphoreType.DMA for _ in range(nbuf)]
        ),
        compiler_params=pltpu.CompilerParams(needs_layout_passes=False),
    )
    def seg_kernel(q_hbm, bsrt_hbm, offs_hbm, cnt_hbm, s_hbm, *scr):
        acc = scr[0]
        rows_bufs = scr[1:1 + nbuf]
        ibuf_s, offs_v, cnt_v = scr[1 + nbuf:4 + nbuf]
        sems = scr[4 + nbuf:]
        w = _worker_id()
        lo = w * NPW
        pltpu.sync_copy(cnt_hbm.at[pl.ds(pl.multiple_of(w * 16, 8), 16)],
                        cnt_v)
        n = cnt_v[pl.ds(0, 16)][0]
        pltpu.sync_copy(offs_hbm.at[pl.ds(pl.multiple_of(w * OFFW, 8), OFFW)],
                        offs_v.at[pl.ds(0, OFFW)])
        nchunks = (n + (g - 1)) // g
        nblocks = (nchunks + (cpb - 1)) // cpb

        neg = jnp.full((16,), _NEG_INF, jnp.float32)

        def init_body(i, _):
            for j in range(nj):
                acc[i, pl.ds(j * 16, 16)] = neg
            return 0
        lax.fori_loop(0, NPW, init_body, 0)

        def compute(rows, gbase, r_in):
            g_valid = jnp.minimum(n - gbase, g)

            def cond(st):
                return st[0] < g_valid

            def body(st):
                e, r = st
                seg_end = offs_v[pl.ds(r + 1, 16)][0] - gbase
                le = jnp.minimum(seg_end, g_valid)
                regs = tuple(acc[r, pl.ds(j * 16, 16)] for j in range(nj))

                def ebody(ei, rg):
                    return tuple(
                        jnp.maximum(rg[j], rows[ei, pl.ds(j * 16, 16)])
                        for j in range(nj))
                regs = lax.fori_loop(e, le, ebody, regs)
                for j in range(nj):
                    acc[r, pl.ds(j * 16, 16)] = regs[j]
                rn = jnp.where(le < g_valid, r + 1, r)
                return (le, rn)

            e_r = lax.while_loop(cond, body, (jnp.int32(0), r_in))
            return e_r[1]

        def gather(c, rows, sem):
            pltpu.async_copy(q_hbm.at[ibuf_s.at[pl.ds(c * g, g)]], rows, sem)

        def wait(rows, sem):
            pltpu.make_async_copy(q_hbm.at[ibuf_s.at[pl.ds(0, g)]],
                                  rows, sem).wait()

        def block_body(ib_i, r_in):
            boff = pl.multiple_of(w * ECAP + ib_i * ib, 8)
            pltpu.sync_copy(bsrt_hbm.at[pl.ds(boff, ib)], ibuf_s)
            ch = jnp.minimum(nchunks - ib_i * cpb, cpb)
            blk_base = ib_i * ib
            for t in range(nbuf - 1):
                @pl.when(t < ch)
                def _(t=t):
                    gather(t, rows_bufs[t], sems[t])

            def grp_body(p, r_c):
                for t in range(nbuf):
                    c = nbuf * p + t
                    tn = (t + nbuf - 1) % nbuf

                    def do(r2, c=c, t=t, tn=tn):
                        wait(rows_bufs[t], sems[t])

                        @pl.when(c + nbuf - 1 < ch)
                        def _():
                            gather(c + nbuf - 1, rows_bufs[tn], sems[tn])
                        return compute(rows_bufs[t], blk_base + c * g, r2)

                    r_c = lax.cond(c < ch, do, lambda r2: r2, r_c)
                return r_c
            return lax.fori_loop(0, (ch + nbuf - 1) // nbuf, grp_body, r_in)
        lax.fori_loop(0, nblocks, block_body, jnp.int32(0))

        pltpu.sync_copy(acc.at[pl.ds(0, NPW)],
                        s_hbm.at[pl.ds(pl.multiple_of(lo, 8), NPW)])

    return seg_kernel


_TR = 1000


def _tc_first(x, A, bias, C, QW):
    cin = x.shape[1]

    def body(x_ref, a_ref, b_ref, p_ref, q_ref):
        r = jnp.dot(x_ref[...], a_ref[...],
                    preferred_element_type=jnp.float32) + b_ref[...]
        p_ref[...] = r[:, :C]
        q = r[:, C:]
        if QW > C:
            q = jnp.concatenate(
                [q, jnp.zeros((q.shape[0], QW - C), jnp.float32)], axis=1)
        q_ref[...] = q

    return pl.pallas_call(
        body,
        grid=(N_NODES // _TR,),
        in_specs=[
            pl.BlockSpec((_TR, cin), lambda i: (i, 0)),
            pl.BlockSpec((cin, 2 * C), lambda i: (0, 0)),
            pl.BlockSpec((1, 2 * C), lambda i: (0, 0)),
        ],
        out_specs=[
            pl.BlockSpec((_TR, C), lambda i: (i, 0)),
            pl.BlockSpec((_TR, QW), lambda i: (i, 0)),
        ],
        out_shape=[jax.ShapeDtypeStruct((N_NODES, C), jnp.float32),
                   jax.ShapeDtypeStruct((N_NODES, QW), jnp.float32)],
    )(x, A, bias)


def _tc_mid(p_prev, s_prev, A, bias, C, split_q=False):
    cin = p_prev.shape[1]
    nq = 2 if split_q else 1
    qw = C // nq

    def body(p_ref, s_ref, a_ref, b_ref, po_ref, *q_refs):
        xv = jnp.maximum(p_ref[...] + s_ref[...], 0.0)
        r = jnp.dot(xv, a_ref[...],
                    preferred_element_type=jnp.float32) + b_ref[...]
        po_ref[...] = r[:, :C]
        for k, q_ref in enumerate(q_refs):
            q_ref[...] = r[:, C + k * qw:C + (k + 1) * qw]

    return pl.pallas_call(
        body,
        grid=(N_NODES // _TR,),
        in_specs=[
            pl.BlockSpec((_TR, cin), lambda i: (i, 0)),
            pl.BlockSpec((_TR, cin), lambda i: (i, 0)),
            pl.BlockSpec((cin, 2 * C), lambda i: (0, 0)),
            pl.BlockSpec((1, 2 * C), lambda i: (0, 0)),
        ],
        out_specs=[pl.BlockSpec((_TR, C), lambda i: (i, 0))]
        + [pl.BlockSpec((_TR, qw), lambda i: (i, 0)) for _ in range(nq)],
        out_shape=[jax.ShapeDtypeStruct((N_NODES, C), jnp.float32)]
        + [jax.ShapeDtypeStruct((N_NODES, qw), jnp.float32)
           for _ in range(nq)],
    )(p_prev, s_prev, A, bias)


def _tc_final(p3, s3a, s3b, x0, W4, b4, W5, b5):
    def body(p_ref, sa_ref, sb_ref, x0_ref, w4_ref, b4_ref, w5_ref, b5_ref,
             o_ref):
        s = jnp.concatenate([sa_ref[...], sb_ref[...]], axis=1)
        xv = jnp.maximum(p_ref[...] + s, 0.0)
        h = jnp.maximum(
            jnp.dot(xv, w4_ref[...], preferred_element_type=jnp.float32)
            + b4_ref[...], 0.0)
        o_ref[...] = (jnp.dot(h, w5_ref[...],
                              preferred_element_type=jnp.float32)
                      + b5_ref[...] + x0_ref[...])

    return pl.pallas_call(
        body,
        grid=(N_NODES // _TR,),
        in_specs=[
            pl.BlockSpec((_TR, 512), lambda i: (i, 0)),
            pl.BlockSpec((_TR, 256), lambda i: (i, 0)),
            pl.BlockSpec((_TR, 256), lambda i: (i, 0)),
            pl.BlockSpec((_TR, 3), lambda i: (i, 0)),
            pl.BlockSpec((512, 256), lambda i: (0, 0)),
            pl.BlockSpec((1, 256), lambda i: (0, 0)),
            pl.BlockSpec((256, 3), lambda i: (0, 0)),
            pl.BlockSpec((1, 3), lambda i: (0, 0)),
        ],
        out_specs=pl.BlockSpec((_TR, 3), lambda i: (i, 0)),
        out_shape=jax.ShapeDtypeStruct((N_NODES, 3), jnp.float32),
    )(p3, s3a, s3b, x0, W4, b4, W5, b5)


def _split_weights(W, b, cin):
    wa, wb = W[:cin], W[cin:]
    A = jnp.concatenate([wa - wb, wb], axis=1)
    bias = jnp.concatenate([b, jnp.zeros_like(b)])[None, :]
    return A, bias


def kernel(x, edge_index, W1, b1, W2, b2, W3, b3, W4, b4, W5, b5):
    src = edge_index[0]
    dst = edge_index[1]

    bsrc, bloc, counts, bsrt, offs = _get_bin_kernel()(src, dst)

    A1, bias1 = _split_weights(W1, b1, 3)
    A2, bias2 = _split_weights(W2, b2, 64)
    A3, bias3 = _split_weights(W3, b3, 128)

    P1, Q1 = _tc_first(x, A1, bias1, 64, 128)
    S1 = _get_segmax(64, 128)(Q1, bsrt, offs, counts)[:N_NODES]

    P2, Q2 = _tc_mid(P1, S1, A2, bias2, 128)
    S2 = _get_segmax(128)(Q2, bsrt, offs, counts)[:N_NODES]

    P3, Q3a, Q3b = _tc_mid(P2, S2, A3, bias3, 512, split_q=True)
    S3a = _get_segmax(256)(Q3a, bsrt, offs, counts)[:N_NODES]
    S3b = _get_segmax(256)(Q3b, bsrt, offs, counts)[:N_NODES]

    return _tc_final(P3, S3a, S3b, x, W4, b4[None, :], W5, b5[None, :])


# speedup vs baseline: 2.5130x; 1.0866x over previous
import functools

import jax
import jax.numpy as jnp
from jax import lax
from jax.experimental import pallas as pl
from jax.experimental.pallas import tpu as pltpu
from jax.experimental.pallas import tpu_sc as plsc

N_NODES = 10000
N_EDGES = 160000

NC = 2
NS = 16
NW = NC * NS
NPW = 320
NPAD = NW * NPW

K_FLUSH = 4096
G = 128
ECAP = N_EDGES + K_FLUSH + 256
SCH = 8000
BUFCAP = K_FLUSH + 256
CAPV = 16384
OFFW = 352

_NEG_INF = float("-inf")


def _worker_id():
    return lax.axis_index("s") * NC + lax.axis_index("c")


def _sc_mesh():
    return plsc.VectorSubcoreMesh(
        core_axis_name="c", subcore_axis_name="s",
        num_cores=NC, num_subcores=NS)


@functools.lru_cache(maxsize=None)
def _get_bin_kernel():
    @functools.partial(
        pl.kernel,
        out_type=[
            jax.ShapeDtypeStruct((NW * ECAP,), jnp.int32),
            jax.ShapeDtypeStruct((NW * ECAP,), jnp.int32),
            jax.ShapeDtypeStruct((NW * 16,), jnp.int32),
            jax.ShapeDtypeStruct((NW * ECAP,), jnp.int32),
            jax.ShapeDtypeStruct((NW * OFFW,), jnp.int32),
        ],
        mesh=_sc_mesh(),
        scratch_types=[
            pltpu.VMEM((SCH,), jnp.int32),
            pltpu.VMEM((SCH,), jnp.int32),
            pltpu.VMEM((SCH,), jnp.int32),
            pltpu.VMEM((SCH,), jnp.int32),
            pltpu.VMEM((BUFCAP,), jnp.int32),
            pltpu.VMEM((BUFCAP,), jnp.int32),
            pltpu.VMEM((16,), jnp.int32),
            pltpu.VMEM((OFFW,), jnp.int32),
            pltpu.VMEM((OFFW,), jnp.int32),
            pltpu.VMEM((OFFW,), jnp.int32),
            pltpu.VMEM((CAPV,), jnp.int32),
            pltpu.SemaphoreType.DMA,
            pltpu.SemaphoreType.DMA,
        ],
        compiler_params=pltpu.CompilerParams(needs_layout_passes=False),
    )
    def bin_edges(src_hbm, dst_hbm, bsrc_hbm, bloc_hbm, cnt_hbm,
                  bsrt_hbm, offs_hbm,
                  stage_sa, stage_da, stage_sb, stage_db,
                  buf_s, buf_l, cnt_v, cnts_v, offs_v, offs2_v, sort_v,
                  sem_a, sem_b):
        w = _worker_id()
        lo = w * NPW

        def do_flush(pos, flushed):
            off = pl.multiple_of(w * ECAP + flushed, 8)
            pltpu.sync_copy(buf_s.at[pl.ds(0, K_FLUSH)],
                            bsrc_hbm.at[pl.ds(off, K_FLUSH)])
            pltpu.sync_copy(buf_l.at[pl.ds(0, K_FLUSH)],
                            bloc_hbm.at[pl.ds(off, K_FLUSH)])
            ts = buf_s[pl.ds(K_FLUSH, 16)]
            tl = buf_l[pl.ds(K_FLUSH, 16)]
            buf_s[pl.ds(0, 16)] = ts
            buf_l[pl.ds(0, 16)] = tl
            return pos - K_FLUSH, flushed + K_FLUSH

        def no_flush(pos, flushed):
            return pos, flushed

        lo_v = jnp.full((16,), lo, jnp.int32)
        hi_v = jnp.full((16,), lo + NPW, jnp.int32)
        zero_v = jnp.zeros((16,), jnp.int32)
        one_v = jnp.full((16,), 1, jnp.int32)

        def make_append(ss, dd):
            def append_chunk(i, carry):
                pos, flushed = carry
                ds_ = [dd[pl.ds(i * 64 + 16 * k, 16)] for k in range(4)]
                ss_ = [ss[pl.ds(i * 64 + 16 * k, 16)] for k in range(4)]
                ms = [(d >= lo_v) & (d < hi_v) for d in ds_]
                css = [plsc.cumsum(jnp.where(m, one_v, zero_v)) for m in ms]
                pos_v = jnp.full((16,), pos, jnp.int32)
                base = zero_v
                tot = jnp.int32(0)
                for k in range(4):
                    idxk = jnp.maximum(pos_v + base + css[k] - one_v, zero_v)
                    plsc.store_scatter(buf_s, [idxk], ss_[k], mask=ms[k])
                    plsc.store_scatter(buf_l, [idxk], ds_[k] - lo_v,
                                       mask=ms[k])
                    ck = css[k][15]
                    base = base + ck
                    tot = tot + ck
                pos = pos + tot
                return lax.cond(pos >= K_FLUSH, do_flush, no_flush,
                                pos, flushed)
            return append_chunk

        nb = N_EDGES // SCH
        bufs = [(stage_sa, stage_da, sem_a), (stage_sb, stage_db, sem_b)]

        def issue(cb, b):
            ss, dd, sem = bufs[b]
            pltpu.async_copy(src_hbm.at[pl.ds(cb * SCH, SCH)], ss, sem)
            pltpu.async_copy(dst_hbm.at[pl.ds(cb * SCH, SCH)], dd, sem)

        def drain(b):
            ss, dd, sem = bufs[b]
            pltpu.make_async_copy(src_hbm.at[pl.ds(0, SCH)], ss, sem).wait()
            pltpu.make_async_copy(src_hbm.at[pl.ds(0, SCH)], dd, sem).wait()

        issue(0, 0)
        carry = (jnp.int32(0), jnp.int32(0))
        for cb in range(nb):
            b = cb % 2
            drain(b)
            if cb + 1 < nb:
                issue(cb + 1, 1 - b)
            ss, dd, _ = bufs[b]
            carry = lax.fori_loop(0, SCH // 64, make_append(ss, dd), carry)
        pos, flushed = carry
        n_total = flushed + pos

        zeros16 = jnp.zeros((16,), jnp.int32)
        pad16 = jnp.full((16,), NPW, jnp.int32)
        for j in range(G // 16):
            buf_s[pl.ds(pos + j * 16, 16)] = zeros16
            buf_l[pl.ds(pos + j * 16, 16)] = pad16
        pos = pos + G
        pos, flushed = lax.cond(pos >= K_FLUSH, do_flush, no_flush,
                                pos, flushed)

        off = pl.multiple_of(w * ECAP + flushed, 8)
        pltpu.sync_copy(buf_s.at[pl.ds(0, K_FLUSH)],
                        bsrc_hbm.at[pl.ds(off, K_FLUSH)])
        pltpu.sync_copy(buf_l.at[pl.ds(0, K_FLUSH)],
                        bloc_hbm.at[pl.ds(off, K_FLUSH)])

        cnt_v[pl.ds(0, 16)] = jnp.full((16,), n_total, jnp.int32)
        pltpu.sync_copy(cnt_v, cnt_hbm.at[pl.ds(pl.multiple_of(w * 16, 8), 16)])

        n = n_total
        ones_v = jnp.full((16,), 1, jnp.int32)
        lanes = lax.broadcasted_iota(jnp.int32, (16,), 0)
        onehots = [lanes == jnp.int32(t) for t in range(16)]

        for k in range(OFFW // 16):
            cnts_v[pl.ds(k * 16, 16)] = zero_v

        nbb = (n + (SCH - 1)) // SCH

        def bcount(bi, _):
            boff = pl.multiple_of(w * ECAP + bi * SCH, 8)
            pltpu.sync_copy(bloc_hbm.at[pl.ds(boff, SCH)], stage_da)
            nv = jnp.minimum(n - bi * SCH, SCH)
            ngrp = (nv + 15) // 16

            def grp(i2, _):
                lv = stage_da[pl.ds(i2 * 16, 16)]
                for t in range(16):
                    plsc.addupdate_scatter(cnts_v, [lv], ones_v,
                                           mask=onehots[t])
                return 0
            lax.fori_loop(0, ngrp, grp, 0)
            return 0
        lax.fori_loop(0, nbb, bcount, 0)

        carry = jnp.int32(0)
        for k in range(OFFW // 16):
            v = cnts_v[pl.ds(k * 16, 16)]
            cs = plsc.cumsum(v)
            offs_v[pl.ds(k * 16, 16)] = jnp.full((16,), carry, jnp.int32) + cs - v
            carry = carry + cs[15]
        pltpu.sync_copy(offs_v.at[pl.ds(0, OFFW)],
                        offs_hbm.at[pl.ds(pl.multiple_of(w * OFFW, 8), OFFW)])

        nrounds = (n + (CAPV - 1)) // CAPV

        def round_body(rd, _):
            win_lo = rd * CAPV
            win_lo_v = jnp.full((16,), win_lo, jnp.int32)

            def zbody(k2, _):
                sort_v[pl.ds(k2 * 16, 16)] = zero_v
                return 0
            lax.fori_loop(0, CAPV // 16, zbody, 0)
            for k in range(OFFW // 16):
                offs2_v[pl.ds(k * 16, 16)] = offs_v[pl.ds(k * 16, 16)]

            def cblk(bi, _):
                boff = pl.multiple_of(w * ECAP + bi * SCH, 8)
                pltpu.sync_copy(bloc_hbm.at[pl.ds(boff, SCH)], stage_da)
                pltpu.sync_copy(bsrc_hbm.at[pl.ds(boff, SCH)], stage_sa)
                nv = jnp.minimum(n - bi * SCH, SCH)
                ngrp = (nv + 15) // 16

                def grp(i2, _):
                    lv = stage_da[pl.ds(i2 * 16, 16)]
                    sv = stage_sa[pl.ds(i2 * 16, 16)]
                    base_e = bi * SCH + i2 * 16
                    for t in range(16):
                        ov = plsc.load_gather(offs2_v, [lv])
                        o = ov[t]
                        dst_i = jnp.clip(o - win_lo, 0, CAPV - 1)
                        valid = ((base_e + t < n) & (o >= win_lo)
                                 & (o < win_lo + CAPV))
                        mvec = onehots[t] & jnp.full((16,), valid, jnp.bool_)
                        plsc.store_scatter(sort_v,
                                           [jnp.full((16,), dst_i, jnp.int32)],
                                           sv, mask=mvec)
                        plsc.addupdate_scatter(offs2_v, [lv], ones_v,
                                               mask=onehots[t])
                    return 0
                lax.fori_loop(0, ngrp, grp, 0)
                return 0
            lax.fori_loop(0, nbb, cblk, 0)

            pltpu.sync_copy(
                sort_v.at[pl.ds(0, CAPV)],
                bsrt_hbm.at[pl.ds(pl.multiple_of(w * ECAP + win_lo, 8),
                                  CAPV)])
            return 0
        lax.fori_loop(0, nrounds, round_body, 0)

        n_rounded = ((n + 15) // 16) * 16

        def zbody2(k2, _):
            sort_v[pl.ds(k2 * 16, 16)] = zero_v
            return 0
        lax.fori_loop(0, G // 16, zbody2, 0)
        pltpu.sync_copy(
            sort_v.at[pl.ds(0, G)],
            bsrt_hbm.at[pl.ds(pl.multiple_of(w * ECAP + n_rounded, 8), G)])

    return bin_edges


IB = 8192


@functools.lru_cache(maxsize=None)
def _get_segmax(C, QW=None):
    QW = QW or C
    if C > 128:
        g, nbuf, ib = 48, 3, 7680
    else:
        g, nbuf, ib = 128, 4, 8192
    cpb = ib // g
    nj = C // 16

    @functools.partial(
        pl.kernel,
        out_type=jax.ShapeDtypeStruct((NPAD, C), jnp.float32),
        mesh=_sc_mesh(),
        scratch_types=(
            [pltpu.VMEM((NPW, C), jnp.float32)]
            + [pltpu.VMEM((g, QW), jnp.float32) for _ in range(nbuf)]
            + [
                pltpu.VMEM((ib,), jnp.int32),
                pltpu.VMEM((OFFW + 16,), jnp.int32),
                pltpu.VMEM((16,), jnp.int32),
            ]
            + [pltpu.SemaphoreType.DMA for _ in range(nbuf)]
        ),
        compiler_params=pltpu.CompilerParams(needs_layout_passes=False),
    )
    def seg_kernel(q_hbm, bsrt_hbm, offs_hbm, cnt_hbm, s_hbm, *scr):
        acc = scr[0]
        rows_bufs = scr[1:1 + nbuf]
        ibuf_s, offs_v, cnt_v = scr[1 + nbuf:4 + nbuf]
        sems = scr[4 + nbuf:]
        w = _worker_id()
        lo = w * NPW
        pltpu.sync_copy(cnt_hbm.at[pl.ds(pl.multiple_of(w * 16, 8), 16)],
                        cnt_v)
        n = cnt_v[pl.ds(0, 16)][0]
        pltpu.sync_copy(offs_hbm.at[pl.ds(pl.multiple_of(w * OFFW, 8), OFFW)],
                        offs_v.at[pl.ds(0, OFFW)])
        nchunks = (n + (g - 1)) // g
        nblocks = (nchunks + (cpb - 1)) // cpb

        neg = jnp.full((16,), _NEG_INF, jnp.float32)

        def init_body(i, _):
            for j in range(nj):
                acc[i, pl.ds(j * 16, 16)] = neg
            return 0
        lax.fori_loop(0, NPW, init_body, 0)

        def compute(rows, gbase, r_in):
            g_valid = jnp.minimum(n - gbase, g)

            def cond(st):
                return st[0] < g_valid

            def body(st):
                e, r = st
                seg_end = offs_v[pl.ds(r + 1, 16)][0] - gbase
                le = jnp.minimum(seg_end, g_valid)
                regs = tuple(acc[r, pl.ds(j * 16, 16)] for j in range(nj))

                def ebody(ei, rg):
                    return tuple(
                        jnp.maximum(rg[j], rows[ei, pl.ds(j * 16, 16)])
                        for j in range(nj))
                regs = lax.fori_loop(e, le, ebody, regs)
                for j in range(nj):
                    acc[r, pl.ds(j * 16, 16)] = regs[j]
                rn = jnp.where(le < g_valid, r + 1, r)
                return (le, rn)

            e_r = lax.while_loop(cond, body, (jnp.int32(0), r_in))
            return e_r[1]

        def gather(c, rows, sem):
            pltpu.async_copy(q_hbm.at[ibuf_s.at[pl.ds(c * g, g)]], rows, sem)

        def wait(rows, sem):
            pltpu.make_async_copy(q_hbm.at[ibuf_s.at[pl.ds(0, g)]],
                                  rows, sem).wait()

        def block_body(ib_i, r_in):
            boff = pl.multiple_of(w * ECAP + ib_i * ib, 8)
            pltpu.sync_copy(bsrt_hbm.at[pl.ds(boff, ib)], ibuf_s)
            ch = jnp.minimum(nchunks - ib_i * cpb, cpb)
            blk_base = ib_i * ib
            for t in range(nbuf - 1):
                @pl.when(t < ch)
                def _(t=t):
                    gather(t, rows_bufs[t], sems[t])

            def grp_body(p, r_c):
                for t in range(nbuf):
                    c = nbuf * p + t
                    tn = (t + nbuf - 1) % nbuf

                    def do(r2, c=c, t=t, tn=tn):
                        wait(rows_bufs[t], sems[t])

                        @pl.when(c + nbuf - 1 < ch)
                        def _():
                            gather(c + nbuf - 1, rows_bufs[tn], sems[tn])
                        return compute(rows_bufs[t], blk_base + c * g, r2)

                    r_c = lax.cond(c < ch, do, lambda r2: r2, r_c)
                return r_c
            return lax.fori_loop(0, (ch + nbuf - 1) // nbuf, grp_body, r_in)
        lax.fori_loop(0, nblocks, block_body, jnp.int32(0))

        pltpu.sync_copy(acc.at[pl.ds(0, NPW)],
                        s_hbm.at[pl.ds(pl.multiple_of(lo, 8), NPW)])

    return seg_kernel


_TR = 1000


def _tc_first(x, A, bias, C, QW):
    cin = x.shape[1]

    def body(x_ref, a_ref, b_ref, p_ref, q_ref):
        r = jnp.dot(x_ref[...], a_ref[...],
                    preferred_element_type=jnp.float32) + b_ref[...]
        p_ref[...] = r[:, :C]
        q = r[:, C:]
        if QW > C:
            q = jnp.concatenate(
                [q, jnp.zeros((q.shape[0], QW - C), jnp.float32)], axis=1)
        q_ref[...] = q

    return pl.pallas_call(
        body,
        grid=(N_NODES // _TR,),
        in_specs=[
            pl.BlockSpec((_TR, cin), lambda i: (i, 0)),
            pl.BlockSpec((cin, 2 * C), lambda i: (0, 0)),
            pl.BlockSpec((1, 2 * C), lambda i: (0, 0)),
        ],
        out_specs=[
            pl.BlockSpec((_TR, C), lambda i: (i, 0)),
            pl.BlockSpec((_TR, QW), lambda i: (i, 0)),
        ],
        out_shape=[jax.ShapeDtypeStruct((N_NODES, C), jnp.float32),
                   jax.ShapeDtypeStruct((N_NODES, QW), jnp.float32)],
    )(x, A, bias)


def _tc_mid(p_prev, s_prev, A, bias, C, split_q=False):
    cin = p_prev.shape[1]
    nq = 2 if split_q else 1
    qw = C // nq

    def body(p_ref, s_ref, a_ref, b_ref, po_ref, *q_refs):
        xv = jnp.maximum(p_ref[...] + s_ref[...], 0.0)
        r = jnp.dot(xv, a_ref[...],
                    preferred_element_type=jnp.float32) + b_ref[...]
        po_ref[...] = r[:, :C]
        for k, q_ref in enumerate(q_refs):
            q_ref[...] = r[:, C + k * qw:C + (k + 1) * qw]

    return pl.pallas_call(
        body,
        grid=(N_NODES // _TR,),
        in_specs=[
            pl.BlockSpec((_TR, cin), lambda i: (i, 0)),
            pl.BlockSpec((_TR, cin), lambda i: (i, 0)),
            pl.BlockSpec((cin, 2 * C), lambda i: (0, 0)),
            pl.BlockSpec((1, 2 * C), lambda i: (0, 0)),
        ],
        out_specs=[pl.BlockSpec((_TR, C), lambda i: (i, 0))]
        + [pl.BlockSpec((_TR, qw), lambda i: (i, 0)) for _ in range(nq)],
        out_shape=[jax.ShapeDtypeStruct((N_NODES, C), jnp.float32)]
        + [jax.ShapeDtypeStruct((N_NODES, qw), jnp.float32)
           for _ in range(nq)],
    )(p_prev, s_prev, A, bias)


def _tc_final(p3, s3a, s3b, x0, W4, b4, W5, b5):
    def body(p_ref, sa_ref, sb_ref, x0_ref, w4_ref, b4_ref, w5_ref, b5_ref,
             o_ref):
        s = jnp.concatenate([sa_ref[...], sb_ref[...]], axis=1)
        xv = jnp.maximum(p_ref[...] + s, 0.0)
        h = jnp.maximum(
            jnp.dot(xv, w4_ref[...], preferred_element_type=jnp.float32)
            + b4_ref[...], 0.0)
        o_ref[...] = (jnp.dot(h, w5_ref[...],
                              preferred_element_type=jnp.float32)
                      + b5_ref[...] + x0_ref[...])

    return pl.pallas_call(
        body,
        grid=(N_NODES // _TR,),
        in_specs=[
            pl.BlockSpec((_TR, 512), lambda i: (i, 0)),
            pl.BlockSpec((_TR, 256), lambda i: (i, 0)),
            pl.BlockSpec((_TR, 256), lambda i: (i, 0)),
            pl.BlockSpec((_TR, 3), lambda i: (i, 0)),
            pl.BlockSpec((512, 256), lambda i: (0, 0)),
            pl.BlockSpec((1, 256), lambda i: (0, 0)),
            pl.BlockSpec((256, 3), lambda i: (0, 0)),
            pl.BlockSpec((1, 3), lambda i: (0, 0)),
        ],
        out_specs=pl.BlockSpec((_TR, 3), lambda i: (i, 0)),
        out_shape=jax.ShapeDtypeStruct((N_NODES, 3), jnp.float32),
    )(p3, s3a, s3b, x0, W4, b4, W5, b5)


def _split_weights(W, b, cin):
    wa, wb = W[:cin], W[cin:]
    A = jnp.concatenate([wa - wb, wb], axis=1)
    bias = jnp.concatenate([b, jnp.zeros_like(b)])[None, :]
    return A, bias


def kernel(x, edge_index, W1, b1, W2, b2, W3, b3, W4, b4, W5, b5):
    src = edge_index[0]
    dst = edge_index[1]

    bsrc, bloc, counts, bsrt, offs = _get_bin_kernel()(src, dst)

    A1, bias1 = _split_weights(W1, b1, 3)
    A2, bias2 = _split_weights(W2, b2, 64)
    A3, bias3 = _split_weights(W3, b3, 128)

    P1, Q1 = _tc_first(x, A1, bias1, 64, 128)
    S1 = _get_segmax(64, 128)(Q1, bsrt, offs, counts)[:N_NODES]

    P2, Q2 = _tc_mid(P1, S1, A2, bias2, 128)
    S2 = _get_segmax(128)(Q2, bsrt, offs, counts)[:N_NODES]

    P3, Q3a, Q3b = _tc_mid(P2, S2, A3, bias3, 512, split_q=True)
    S3a = _get_segmax(256)(Q3a, bsrt, offs, counts)[:N_NODES]
    S3b = _get_segmax(256)(Q3b, bsrt, offs, counts)[:N_NODES]

    return _tc_final(P3, S3a, S3b, x, W4, b4[None, :], W5, b5[None, :])
